# Initial kernel scaffold; baseline (speedup 1.0000x reference)
import functools

import jax
import jax.numpy as jnp
from jax import lax
from jax.experimental import pallas as pl
from jax.experimental.pallas import tpu as pltpu
from jax.experimental.pallas import tpu_sc as plsc

NU = 30000
NI = 70000
D = 64
NNZ = 1000000
NB = 4096
NLAYERS = 3.0
DECAY = 1e-4

NC = 2
NS = 16
CH = 800
EPAD = 1024000
NPAD = EPAD - NNZ
NDUM = 1600
AU = NU + 2000
AI = NI + 2000
EW = EPAD // NS
NCH = EW // CH
EWD = EPAD // (NC * NS)
NCHD = EWD // CH

_mesh = plsc.VectorSubcoreMesh(core_axis_name="c", subcore_axis_name="s",
                               num_cores=NC, num_subcores=NS)


def _sc_spmm_u_body(rows_s_ref, cols_g_ref, item_t_ref, zrow_ref, zdeg_ref,
                    hu_ref, deg_ref,
                    cidx, gidx, ridx, didx, rowsv, onesv, acc, dacc, sem):
    c = lax.axis_index("c")
    s = lax.axis_index("s")

    def _ones(t, _):
        onesv[pl.ds(t * 16, 16)] = jnp.full((16,), 1.0, jnp.float32)
        return 0
    lax.fori_loop(0, CH // 16, _ones, 0)

    pltpu.sync_copy(zrow_ref, acc.at[pl.ds(s * 2000, 2000)])
    pltpu.sync_copy(zdeg_ref, dacc.at[pl.ds(s * 2000, 2000)])
    plsc.subcore_barrier()

    coff = c * NI

    def _chunk(k, _):
        base = s * EW + k * CH
        pltpu.sync_copy(cols_g_ref.at[pl.ds(base, CH)], cidx)

        def _off(t, __):
            gidx[pl.ds(t * 16, 16)] = cidx[pl.ds(t * 16, 16)] + coff
            return 0
        lax.fori_loop(0, CH // 16, _off, 0)
        pltpu.async_copy(item_t_ref.at[gidx], rowsv, sem).wait()
        pltpu.sync_copy(rows_s_ref.at[pl.ds(base, CH)], ridx)
        pltpu.sync_copy(rowsv, acc.at[ridx], add=True)
        return 0
    lax.fori_loop(0, NCH, _chunk, 0)

    def _dchunk(k, _):
        base = (c * NS + s) * EWD + k * CH
        pltpu.sync_copy(rows_s_ref.at[pl.ds(base, CH)], didx)
        pltpu.sync_copy(onesv, dacc.at[didx], add=True)
        return 0
    lax.fori_loop(0, NCHD, _dchunk, 0)

    plsc.subcore_barrier()
    pltpu.sync_copy(acc.at[pl.ds(s * 2000, 2000)],
                    hu_ref.at[pl.ds(c * AU + s * 2000, 2000)])
    pltpu.sync_copy(dacc.at[pl.ds(s * 2000, 2000)],
                    deg_ref.at[pl.ds(c * AU + s * 2000, 2000)])


_sc_spmm_u = functools.partial(
    pl.kernel,
    out_type=[jax.ShapeDtypeStruct((NC * AU, 32), jnp.float32),
              jax.ShapeDtypeStruct((NC * AU,), jnp.float32)],
    mesh=_mesh,
    scratch_types=[
        pltpu.VMEM((CH,), jnp.int32),
        pltpu.VMEM((CH,), jnp.int32),
        pltpu.VMEM((CH,), jnp.int32),
        pltpu.VMEM((CH,), jnp.int32),
        pltpu.VMEM((CH, 32), jnp.float32),
        pltpu.VMEM((CH,), jnp.float32),
        pltpu.VMEM_SHARED((AU, 32), jnp.float32),
        pltpu.VMEM_SHARED((AU,), jnp.float32),
        pltpu.SemaphoreType.DMA,
    ],
)(_sc_spmm_u_body)


def _sc_spmm_i_body(rows_g_ref, cols_s_ref, su_ref, zrow_ref,
                    hi_ref,
                    cidx, gidx, ridx, rowsv, acc, sem):
    c = lax.axis_index("c")
    s = lax.axis_index("s")
    for j in range(2):
        sl = c * 2 + j
        pltpu.sync_copy(zrow_ref, acc.at[pl.ds(s * 4500, 4500)])
        plsc.subcore_barrier()
        soff = sl * NU

        def _chunk(k, _):
            base = s * EW + k * CH
            pltpu.sync_copy(rows_g_ref.at[pl.ds(base, CH)], ridx)

            def _off(t, __):
                gidx[pl.ds(t * 16, 16)] = ridx[pl.ds(t * 16, 16)] + soff
                return 0
            lax.fori_loop(0, CH // 16, _off, 0)
            pltpu.async_copy(su_ref.at[gidx], rowsv, sem).wait()
            pltpu.sync_copy(cols_s_ref.at[pl.ds(base, CH)], cidx)
            pltpu.sync_copy(rowsv, acc.at[cidx], add=True)
            return 0
        lax.fori_loop(0, NCH, _chunk, 0)

        plsc.subcore_barrier()
        pltpu.sync_copy(acc.at[pl.ds(s * 4500, 4500)],
                        hi_ref.at[pl.ds(sl * AI + s * 4500, 4500)])


_sc_spmm_i = functools.partial(
    pl.kernel,
    out_type=jax.ShapeDtypeStruct((4 * AI, 16), jnp.float32),
    mesh=_mesh,
    scratch_types=[
        pltpu.VMEM((CH,), jnp.int32),
        pltpu.VMEM((CH,), jnp.int32),
        pltpu.VMEM((CH,), jnp.int32),
        pltpu.VMEM((CH, 16), jnp.float32),
        pltpu.VMEM_SHARED((AI, 16), jnp.float32),
        pltpu.SemaphoreType.DMA,
    ],
)(_sc_spmm_i_body)


def _sc_gather_body(u_ref, i_ref, bu_ref, bp_ref, bn_ref,
                    ug_ref, pg_ref, ng_ref,
                    idxv, rowsv, sem):
    c = lax.axis_index("c")
    s = lax.axis_index("s")
    n = NB // (NC * NS)
    base = (s * NC + c) * n
    for src, idx_hbm, out in ((u_ref, bu_ref, ug_ref),
                              (i_ref, bp_ref, pg_ref),
                              (i_ref, bn_ref, ng_ref)):
        pltpu.sync_copy(idx_hbm.at[pl.ds(base, n)], idxv)
        pltpu.async_copy(src.at[idxv], rowsv, sem).wait()
        pltpu.sync_copy(rowsv, out.at[pl.ds(base, n)])


_sc_gather = functools.partial(
    pl.kernel,
    out_type=[jax.ShapeDtypeStruct((NB, D), jnp.float32)] * 3,
    mesh=_mesh,
    scratch_types=[
        pltpu.VMEM((NB // (NC * NS),), jnp.int32),
        pltpu.VMEM((NB // (NC * NS), D), jnp.float32),
        pltpu.SemaphoreType.DMA,
    ],
)(_sc_gather_body)


def _t1_body(re_ref, mask_ref, ent_ref, re2_ref, mask2_ref, ent2_ref, w2_ref,
             rfi_ref, rfu_ref):
    def _rel(r, m, e):
        x = r[...]
        x = jnp.exp(x - jnp.max(x, axis=1, keepdims=True))
        sm = x / jnp.sum(x, axis=1, keepdims=True)
        return jnp.dot(sm * m[...], e[...], preferred_element_type=jnp.float32)
    rfi_ref[...] = jnp.dot(_rel(re_ref, mask_ref, ent_ref), w2_ref[...],
                           preferred_element_type=jnp.float32)
    rfu_ref[...] = jnp.dot(_rel(re2_ref, mask2_ref, ent2_ref), w2_ref[...],
                           preferred_element_type=jnp.float32)


def _t1(re, mask, ent, re2, mask2, ent2, w2):
    return pl.pallas_call(
        _t1_body,
        out_shape=[jax.ShapeDtypeStruct((16, D), jnp.float32),
                   jax.ShapeDtypeStruct((8, D), jnp.float32)],
    )(re, mask, ent, re2, mask2, ent2, w2)


_BLK = 2000


def _t2_body(sn_ref, ir_ref, w1_ref, rfi_ref, out_ref):
    out_ref[0] = (
        jnp.dot(sn_ref[...], w1_ref[...], preferred_element_type=jnp.float32)
        + jnp.dot(ir_ref[...], rfi_ref[...], preferred_element_type=jnp.float32))


def _t2(struct_node_emb, ir, w1, rfi):
    nb = NI // _BLK
    return pl.pallas_call(
        _t2_body,
        grid=(2, nb),
        in_specs=[
            pl.BlockSpec((_BLK, D), lambda h, i: (NU // _BLK + i, 0)),
            pl.BlockSpec((_BLK, 16), lambda h, i: (i, 0)),
            pl.BlockSpec((D, 32), lambda h, i: (0, h)),
            pl.BlockSpec((16, 32), lambda h, i: (0, h)),
        ],
        out_specs=pl.BlockSpec((1, _BLK, 32), lambda h, i: (h, i, 0)),
        out_shape=jax.ShapeDtypeStruct((2, NI, 32), jnp.float32),
    )(struct_node_emb, ir, w1, rfi)


def _t3_body(sn_ref, ur_ref, w1_ref, rfu_ref, out_ref):
    out_ref[...] = (
        jnp.dot(sn_ref[...], w1_ref[...], preferred_element_type=jnp.float32)
        + jnp.dot(ur_ref[...], rfu_ref[...], preferred_element_type=jnp.float32))


def _t3(struct_node_emb, ur, w1, rfu):
    nb = NU // _BLK
    return pl.pallas_call(
        _t3_body,
        grid=(nb,),
        in_specs=[
            pl.BlockSpec((_BLK, D), lambda i: (i, 0)),
            pl.BlockSpec((_BLK, 8), lambda i: (i, 0)),
            pl.BlockSpec((D, D), lambda i: (0, 0)),
            pl.BlockSpec((8, D), lambda i: (0, 0)),
        ],
        out_specs=pl.BlockSpec((_BLK, D), lambda i: (i, 0)),
        out_shape=jax.ShapeDtypeStruct((NU, D), jnp.float32),
    )(struct_node_emb, ur, w1, rfu)


def _t4_body(uc_ref, d0_ref, d1_ref, out_ref):
    deg = d0_ref[...] + d1_ref[...]
    dinv = jnp.where(deg > 0, 1.0 / deg, 0.0)
    out_ref[0] = uc_ref[...] * dinv


def _t4(user_comb, deg0, deg1):
    nb = NU // _BLK
    return pl.pallas_call(
        _t4_body,
        grid=(4, nb),
        in_specs=[
            pl.BlockSpec((_BLK, 16), lambda k, i: (i, k)),
            pl.BlockSpec((_BLK, 1), lambda k, i: (i, 0)),
            pl.BlockSpec((_BLK, 1), lambda k, i: (i, 0)),
        ],
        out_specs=pl.BlockSpec((1, _BLK, 16), lambda k, i: (k, i, 0)),
        out_shape=jax.ShapeDtypeStruct((4, NU, 16), jnp.float32),
    )(user_comb, deg0, deg1)


def _finalize(x):
    x = jnp.where(x > 0, x, 0.2 * x)
    n = jnp.sqrt(jnp.sum(x * x, axis=1, keepdims=True))
    return NLAYERS * x / jnp.maximum(n, 1e-12)


def _t5u_body(a_ref, b_ref, d0_ref, d1_ref, bias_ref, out_ref):
    deg = d0_ref[...] + d1_ref[...]
    dinv = jnp.where(deg > 0, 1.0 / deg, 0.0)
    h = jnp.concatenate([a_ref[...], b_ref[...]], axis=1) * dinv + bias_ref[...]
    out_ref[...] = _finalize(h)


def _t5u(hu_a, hu_b, deg0, deg1, bias):
    nb = NU // _BLK
    return pl.pallas_call(
        _t5u_body,
        grid=(nb,),
        in_specs=[
            pl.BlockSpec((_BLK, 32), lambda i: (i, 0)),
            pl.BlockSpec((_BLK, 32), lambda i: (i, 0)),
            pl.BlockSpec((_BLK, 1), lambda i: (i, 0)),
            pl.BlockSpec((_BLK, 1), lambda i: (i, 0)),
            pl.BlockSpec((_BLK, D), lambda i: (i, 0)),
        ],
        out_specs=pl.BlockSpec((_BLK, D), lambda i: (i, 0)),
        out_shape=jax.ShapeDtypeStruct((NU, D), jnp.float32),
    )(hu_a, hu_b, deg0, deg1, bias)


def _t5i_body(h0_ref, h1_ref, h2_ref, h3_ref, bias_ref, out_ref):
    h = jnp.concatenate(
        [h0_ref[...], h1_ref[...], h2_ref[...], h3_ref[...]], axis=1)
    out_ref[...] = _finalize(h + bias_ref[...])


def _t5i(h0, h1, h2, h3, bias):
    nb = NI // _BLK
    return pl.pallas_call(
        _t5i_body,
        grid=(nb,),
        in_specs=[pl.BlockSpec((_BLK, 16), lambda i: (i, 0))] * 4 + [
            pl.BlockSpec((_BLK, D), lambda i: (NU // _BLK + i, 0)),
        ],
        out_specs=pl.BlockSpec((_BLK, D), lambda i: (i, 0)),
        out_shape=jax.ShapeDtypeStruct((NI, D), jnp.float32),
    )(h0, h1, h2, h3, bias)


def _t6_body(ug_ref, pg_ref, ng_ref, out_ref):
    ug, pg, ng = ug_ref[...], pg_ref[...], ng_ref[...]
    ps = jnp.sum(ug * pg, axis=1, keepdims=True)
    ns = jnp.sum(ug * ng, axis=1, keepdims=True)
    x = ps - ns
    ls = jnp.minimum(x, 0.0) - jnp.log(1.0 + jnp.exp(-jnp.abs(x)))
    mf = -jnp.sum(ls) / NB
    reg = (jnp.sum(ug * ug) + jnp.sum(pg * pg) + jnp.sum(ng * ng)) * 0.5
    out_ref[0, 0] = mf + DECAY * reg / NB


def _t6(ug, pg, ng):
    return pl.pallas_call(
        _t6_body,
        out_shape=jax.ShapeDtypeStruct((1, 1), jnp.float32),
    )(ug, pg, ng)


def kernel(struct_node_emb, train_weight, train_weight_2, bias, re, entity_emb,
           ir, re_2, entity_emb_2, ur, mask, mask_2, rows, cols,
           batch_users, batch_pos, batch_neg):
    rows = rows.astype(jnp.int32)
    cols = cols.astype(jnp.int32)
    pad = jnp.arange(NPAD, dtype=jnp.int32)
    rows_s = jnp.concatenate([rows, NU + pad % NDUM])
    rows_g = jnp.concatenate([rows, pad % 16])
    cols_s = jnp.concatenate([cols, NI + pad % NDUM])
    cols_g = jnp.concatenate([cols, pad % 16])

    rfi, rfu = _t1(re, mask, entity_emb, re_2, mask_2, entity_emb_2,
                   train_weight_2)
    item_t = _t2(struct_node_emb, ir, train_weight, rfi).reshape(2 * NI, 32)
    user_comb = _t3(struct_node_emb, ur, train_weight, rfu)

    z2d = jnp.zeros((2000, 32), jnp.float32)
    z1d = jnp.zeros((2000,), jnp.float32)
    hu_flat, deg_flat = _sc_spmm_u(rows_s, cols_g, item_t, z2d, z1d)

    deg0 = deg_flat[:NU].reshape(NU, 1)
    deg1 = deg_flat[AU:AU + NU].reshape(NU, 1)
    su = _t4(user_comb, deg0, deg1).reshape(4 * NU, 16)

    z2i = jnp.zeros((4500, 16), jnp.float32)
    hi_flat = _sc_spmm_i(rows_g, cols_s, su, z2i)

    u = _t5u(hu_flat[:NU], hu_flat[AU:AU + NU], deg0, deg1, bias)
    iv = _t5i(hi_flat[:NI], hi_flat[AI:AI + NI], hi_flat[2 * AI:2 * AI + NI],
              hi_flat[3 * AI:3 * AI + NI], bias)

    ug, pg, ng = _sc_gather(u, iv, batch_users.astype(jnp.int32),
                            batch_pos.astype(jnp.int32),
                            batch_neg.astype(jnp.int32))
    loss = _t6(ug, pg, ng).reshape(())
    return (loss, u, iv)


# speedup vs baseline: 13.9423x; 13.9423x over previous
import functools

import jax
import jax.numpy as jnp
from jax import lax
from jax.experimental import pallas as pl
from jax.experimental.pallas import tpu as pltpu
from jax.experimental.pallas import tpu_sc as plsc

NU = 30000
NI = 70000
D = 64
NNZ = 1000000
NB = 4096
NLAYERS = 3.0
DECAY = 1e-4

NC = 2
NS = 16
CH = 800
EPAD = 1024000
NPAD = EPAD - NNZ
NDUM = 1600
AU = NU + 2000
AI = NI + 2000
EW = EPAD // NS
NCH = EW // CH
EWD = EPAD // (NC * NS)
NCHD = EWD // CH

_mesh = plsc.VectorSubcoreMesh(core_axis_name="c", subcore_axis_name="s",
                               num_cores=NC, num_subcores=NS)
_sc_params = pltpu.CompilerParams(use_tc_tiling_on_sc=False)


def _sc_spmm_u_body(rows_s_ref, cols_g_ref, item_t_ref, zrow_ref, zdeg_ref,
                    hu_ref, deg_ref,
                    cidx, gidx, ridx, didx, rowsv, onesv, zrow_v, zdeg_v,
                    acc, dacc, sem):
    c = lax.axis_index("c")
    s = lax.axis_index("s")

    def _ones(t, _):
        onesv[pl.ds(t * 16, 16)] = jnp.full((16,), 1.0, jnp.float32)
        return 0
    lax.fori_loop(0, CH // 16, _ones, 0)

    pltpu.sync_copy(zrow_ref, zrow_v)
    pltpu.sync_copy(zdeg_ref, zdeg_v)
    for q in range(4):
        pltpu.sync_copy(zrow_v, acc.at[pl.ds(s * 2000 + q * 500, 500)])
    pltpu.sync_copy(zdeg_v, dacc.at[pl.ds(s * 2000, 2000)])
    plsc.subcore_barrier()

    coff = c * NI

    def _chunk(k, _):
        base = s * EW + k * CH
        pltpu.sync_copy(cols_g_ref.at[pl.ds(base, CH)], cidx)

        def _off(t, __):
            gidx[pl.ds(t * 16, 16)] = cidx[pl.ds(t * 16, 16)] + coff
            return 0
        lax.fori_loop(0, CH // 16, _off, 0)
        pltpu.async_copy(item_t_ref.at[gidx], rowsv, sem).wait()
        pltpu.sync_copy(rows_s_ref.at[pl.ds(base, CH)], ridx)
        pltpu.sync_copy(rowsv, acc.at[ridx], add=True)
        return 0
    lax.fori_loop(0, NCH, _chunk, 0)

    def _dchunk(k, _):
        base = (c * NS + s) * EWD + k * CH
        pltpu.sync_copy(rows_s_ref.at[pl.ds(base, CH)], didx)
        pltpu.sync_copy(onesv, dacc.at[didx], add=True)
        return 0
    lax.fori_loop(0, NCHD, _dchunk, 0)

    plsc.subcore_barrier()
    for q in range(4):
        pltpu.sync_copy(acc.at[pl.ds(s * 2000 + q * 500, 500)], zrow_v)
        pltpu.sync_copy(zrow_v, hu_ref.at[pl.ds(c * AU + s * 2000 + q * 500, 500)])
    pltpu.sync_copy(dacc.at[pl.ds(s * 2000, 2000)], zdeg_v)
    pltpu.sync_copy(zdeg_v, deg_ref.at[pl.ds(c * AU + s * 2000, 2000)])


_sc_spmm_u = functools.partial(
    pl.kernel,
    out_type=[jax.ShapeDtypeStruct((NC * AU, 32), jnp.float32),
              jax.ShapeDtypeStruct((NC * AU,), jnp.float32)],
    mesh=_mesh,
    scratch_types=[
        pltpu.VMEM((CH,), jnp.int32),
        pltpu.VMEM((CH,), jnp.int32),
        pltpu.VMEM((CH,), jnp.int32),
        pltpu.VMEM((CH,), jnp.int32),
        pltpu.VMEM((CH, 32), jnp.float32),
        pltpu.VMEM((CH,), jnp.float32),
        pltpu.VMEM((500, 32), jnp.float32),
        pltpu.VMEM((2000,), jnp.float32),
        pltpu.VMEM_SHARED((AU, 32), jnp.float32),
        pltpu.VMEM_SHARED((AU,), jnp.float32),
        pltpu.SemaphoreType.DMA,
    ],
    compiler_params=_sc_params,
)(_sc_spmm_u_body)


def _sc_spmm_i_body(rows_g_ref, cols_s_ref, su_ref, zrow_ref,
                    hi_ref,
                    cidx, gidx, ridx, rowsv, zrow_v, acc, sem):
    c = lax.axis_index("c")
    s = lax.axis_index("s")
    for j in range(2):
        sl = c * 2 + j
        pltpu.sync_copy(zrow_ref, zrow_v)
        for q in range(9):
            pltpu.sync_copy(zrow_v, acc.at[pl.ds(s * 4500 + q * 500, 500)])
        plsc.subcore_barrier()
        soff = sl * NU

        def _chunk(k, _):
            base = s * EW + k * CH
            pltpu.sync_copy(rows_g_ref.at[pl.ds(base, CH)], ridx)

            def _off(t, __):
                gidx[pl.ds(t * 16, 16)] = ridx[pl.ds(t * 16, 16)] + soff
                return 0
            lax.fori_loop(0, CH // 16, _off, 0)
            pltpu.async_copy(su_ref.at[gidx], rowsv, sem).wait()
            pltpu.sync_copy(cols_s_ref.at[pl.ds(base, CH)], cidx)
            pltpu.sync_copy(rowsv, acc.at[cidx], add=True)
            return 0
        lax.fori_loop(0, NCH, _chunk, 0)

        plsc.subcore_barrier()
        for q in range(9):
            pltpu.sync_copy(acc.at[pl.ds(s * 4500 + q * 500, 500)], zrow_v)
            pltpu.sync_copy(
                zrow_v, hi_ref.at[pl.ds(sl * AI + s * 4500 + q * 500, 500)])


_sc_spmm_i = functools.partial(
    pl.kernel,
    out_type=jax.ShapeDtypeStruct((4 * AI, 16), jnp.float32),
    mesh=_mesh,
    scratch_types=[
        pltpu.VMEM((CH,), jnp.int32),
        pltpu.VMEM((CH,), jnp.int32),
        pltpu.VMEM((CH,), jnp.int32),
        pltpu.VMEM((CH, 16), jnp.float32),
        pltpu.VMEM((500, 16), jnp.float32),
        pltpu.VMEM_SHARED((AI, 16), jnp.float32),
        pltpu.SemaphoreType.DMA,
    ],
    compiler_params=_sc_params,
)(_sc_spmm_i_body)


def _sc_gather_body(u_ref, i_ref, bu_ref, bp_ref, bn_ref,
                    ug_ref, pg_ref, ng_ref,
                    idxv, rowsv, sem):
    c = lax.axis_index("c")
    s = lax.axis_index("s")
    n = NB // (NC * NS)
    base = (s * NC + c) * n
    for src, idx_hbm, out in ((u_ref, bu_ref, ug_ref),
                              (i_ref, bp_ref, pg_ref),
                              (i_ref, bn_ref, ng_ref)):
        pltpu.sync_copy(idx_hbm.at[pl.ds(base, n)], idxv)
        pltpu.async_copy(src.at[idxv], rowsv, sem).wait()
        pltpu.sync_copy(rowsv, out.at[pl.ds(base, n)])


_sc_gather = functools.partial(
    pl.kernel,
    out_type=[jax.ShapeDtypeStruct((NB, D), jnp.float32)] * 3,
    mesh=_mesh,
    scratch_types=[
        pltpu.VMEM((NB // (NC * NS),), jnp.int32),
        pltpu.VMEM((NB // (NC * NS), D), jnp.float32),
        pltpu.SemaphoreType.DMA,
    ],
    compiler_params=_sc_params,
)(_sc_gather_body)


def _t1_body(re_ref, mask_ref, ent_ref, re2_ref, mask2_ref, ent2_ref, w2_ref,
             rfi_ref, rfu_ref):
    def _rel(r, m, e):
        x = r[...]
        x = jnp.exp(x - jnp.max(x, axis=1, keepdims=True))
        sm = x / jnp.sum(x, axis=1, keepdims=True)
        return jnp.dot(sm * m[...], e[...], preferred_element_type=jnp.float32)
    rfi_ref[...] = jnp.dot(_rel(re_ref, mask_ref, ent_ref), w2_ref[...],
                           preferred_element_type=jnp.float32)
    rfu_ref[...] = jnp.dot(_rel(re2_ref, mask2_ref, ent2_ref), w2_ref[...],
                           preferred_element_type=jnp.float32)


def _t1(re, mask, ent, re2, mask2, ent2, w2):
    return pl.pallas_call(
        _t1_body,
        out_shape=[jax.ShapeDtypeStruct((16, D), jnp.float32),
                   jax.ShapeDtypeStruct((8, D), jnp.float32)],
    )(re, mask, ent, re2, mask2, ent2, w2)


_BLK = 2000


def _t2_body(sn_ref, ir_ref, w1_ref, rfi_ref, out_ref):
    out_ref[0] = (
        jnp.dot(sn_ref[...], w1_ref[0], preferred_element_type=jnp.float32)
        + jnp.dot(ir_ref[...], rfi_ref[0], preferred_element_type=jnp.float32))


def _t2(struct_node_emb, ir, w1h, rfih):
    nb = NI // _BLK
    return pl.pallas_call(
        _t2_body,
        grid=(2, nb),
        in_specs=[
            pl.BlockSpec((_BLK, D), lambda h, i: (NU // _BLK + i, 0)),
            pl.BlockSpec((_BLK, 16), lambda h, i: (i, 0)),
            pl.BlockSpec((1, D, 32), lambda h, i: (h, 0, 0)),
            pl.BlockSpec((1, 16, 32), lambda h, i: (h, 0, 0)),
        ],
        out_specs=pl.BlockSpec((1, _BLK, 32), lambda h, i: (h, i, 0)),
        out_shape=jax.ShapeDtypeStruct((2, NI, 32), jnp.float32),
    )(struct_node_emb, ir, w1h, rfih)


def _t3_body(sn_ref, ur_ref, w1_ref, rfu_ref, out_ref):
    out_ref[...] = (
        jnp.dot(sn_ref[...], w1_ref[...], preferred_element_type=jnp.float32)
        + jnp.dot(ur_ref[...], rfu_ref[...], preferred_element_type=jnp.float32))


def _t3(struct_node_emb, ur, w1, rfu):
    nb = NU // _BLK
    return pl.pallas_call(
        _t3_body,
        grid=(nb,),
        in_specs=[
            pl.BlockSpec((_BLK, D), lambda i: (i, 0)),
            pl.BlockSpec((_BLK, 8), lambda i: (i, 0)),
            pl.BlockSpec((D, D), lambda i: (0, 0)),
            pl.BlockSpec((8, D), lambda i: (0, 0)),
        ],
        out_specs=pl.BlockSpec((_BLK, D), lambda i: (i, 0)),
        out_shape=jax.ShapeDtypeStruct((NU, D), jnp.float32),
    )(struct_node_emb, ur, w1, rfu)


def _t4_body(uc_ref, d0_ref, d1_ref, o0, o1, o2, o3):
    deg = d0_ref[...] + d1_ref[...]
    dinv = jnp.where(deg > 0, 1.0 / deg, 0.0)
    x = uc_ref[...] * dinv
    o0[...] = x[:, 0:16]
    o1[...] = x[:, 16:32]
    o2[...] = x[:, 32:48]
    o3[...] = x[:, 48:64]


def _t4(user_comb, deg0, deg1):
    nb = NU // _BLK
    return pl.pallas_call(
        _t4_body,
        grid=(nb,),
        in_specs=[
            pl.BlockSpec((_BLK, D), lambda i: (i, 0)),
            pl.BlockSpec((_BLK, 1), lambda i: (i, 0)),
            pl.BlockSpec((_BLK, 1), lambda i: (i, 0)),
        ],
        out_specs=[pl.BlockSpec((_BLK, 16), lambda i: (i, 0))] * 4,
        out_shape=[jax.ShapeDtypeStruct((NU, 16), jnp.float32)] * 4,
    )(user_comb, deg0, deg1)


def _finalize(x):
    x = jnp.where(x > 0, x, 0.2 * x)
    n = jnp.sqrt(jnp.sum(x * x, axis=1, keepdims=True))
    return NLAYERS * x / jnp.maximum(n, 1e-12)


def _t5u_body(a_ref, b_ref, d0_ref, d1_ref, bias_ref, out_ref):
    deg = d0_ref[...] + d1_ref[...]
    dinv = jnp.where(deg > 0, 1.0 / deg, 0.0)
    h = jnp.concatenate([a_ref[...], b_ref[...]], axis=1) * dinv + bias_ref[...]
    out_ref[...] = _finalize(h)


def _t5u(hu_a, hu_b, deg0, deg1, bias):
    nb = NU // _BLK
    return pl.pallas_call(
        _t5u_body,
        grid=(nb,),
        in_specs=[
            pl.BlockSpec((_BLK, 32), lambda i: (i, 0)),
            pl.BlockSpec((_BLK, 32), lambda i: (i, 0)),
            pl.BlockSpec((_BLK, 1), lambda i: (i, 0)),
            pl.BlockSpec((_BLK, 1), lambda i: (i, 0)),
            pl.BlockSpec((_BLK, D), lambda i: (i, 0)),
        ],
        out_specs=pl.BlockSpec((_BLK, D), lambda i: (i, 0)),
        out_shape=jax.ShapeDtypeStruct((NU, D), jnp.float32),
    )(hu_a, hu_b, deg0, deg1, bias)


def _t5i_body(h0_ref, h1_ref, h2_ref, h3_ref, bias_ref, out_ref):
    h = jnp.concatenate(
        [h0_ref[...], h1_ref[...], h2_ref[...], h3_ref[...]], axis=1)
    out_ref[...] = _finalize(h + bias_ref[...])


def _t5i(h0, h1, h2, h3, bias):
    nb = NI // _BLK
    return pl.pallas_call(
        _t5i_body,
        grid=(nb,),
        in_specs=[pl.BlockSpec((_BLK, 16), lambda i: (i, 0))] * 4 + [
            pl.BlockSpec((_BLK, D), lambda i: (NU // _BLK + i, 0)),
        ],
        out_specs=pl.BlockSpec((_BLK, D), lambda i: (i, 0)),
        out_shape=jax.ShapeDtypeStruct((NI, D), jnp.float32),
    )(h0, h1, h2, h3, bias)


def _t6_body(ug_ref, pg_ref, ng_ref, out_ref):
    ug, pg, ng = ug_ref[...], pg_ref[...], ng_ref[...]
    ps = jnp.sum(ug * pg, axis=1, keepdims=True)
    ns = jnp.sum(ug * ng, axis=1, keepdims=True)
    x = ps - ns
    ls = jnp.minimum(x, 0.0) - jnp.log(1.0 + jnp.exp(-jnp.abs(x)))
    mf = -jnp.sum(ls) / NB
    reg = (jnp.sum(ug * ug) + jnp.sum(pg * pg) + jnp.sum(ng * ng)) * 0.5
    out_ref[...] = jnp.reshape(mf + DECAY * reg / NB, (1, 1))


def _t6(ug, pg, ng):
    return pl.pallas_call(
        _t6_body,
        out_shape=jax.ShapeDtypeStruct((1, 1), jnp.float32),
    )(ug, pg, ng)


def kernel(struct_node_emb, train_weight, train_weight_2, bias, re, entity_emb,
           ir, re_2, entity_emb_2, ur, mask, mask_2, rows, cols,
           batch_users, batch_pos, batch_neg):
    rows = rows.astype(jnp.int32)
    cols = cols.astype(jnp.int32)
    pad = jnp.arange(NPAD, dtype=jnp.int32)
    rows_s = jnp.concatenate([rows, NU + pad % NDUM])
    rows_g = jnp.concatenate([rows, pad % 16])
    cols_s = jnp.concatenate([cols, NI + pad % NDUM])
    cols_g = jnp.concatenate([cols, pad % 16])

    rfi, rfu = _t1(re, mask, entity_emb, re_2, mask_2, entity_emb_2,
                   train_weight_2)
    w1h = jnp.stack([train_weight[:, :32], train_weight[:, 32:]])
    rfih = jnp.stack([rfi[:, :32], rfi[:, 32:]])
    item_t = _t2(struct_node_emb, ir, w1h, rfih).reshape(2 * NI, 32)
    user_comb = _t3(struct_node_emb, ur, train_weight, rfu)

    z2d = jnp.zeros((500, 32), jnp.float32)
    z1d = jnp.zeros((2000,), jnp.float32)
    hu_flat, deg_flat = _sc_spmm_u(rows_s, cols_g, item_t, z2d, z1d)

    deg0 = deg_flat[:NU].reshape(NU, 1)
    deg1 = deg_flat[AU:AU + NU].reshape(NU, 1)
    su = jnp.concatenate(_t4(user_comb, deg0, deg1), axis=0)

    z2i = jnp.zeros((500, 16), jnp.float32)
    hi_flat = _sc_spmm_i(rows_g, cols_s, su, z2i)

    u = _t5u(hu_flat[:NU], hu_flat[AU:AU + NU], deg0, deg1, bias)
    iv = _t5i(hi_flat[:NI], hi_flat[AI:AI + NI], hi_flat[2 * AI:2 * AI + NI],
              hi_flat[3 * AI:3 * AI + NI], bias)

    ug, pg, ng = _sc_gather(u, iv, batch_users.astype(jnp.int32),
                            batch_pos.astype(jnp.int32),
                            batch_neg.astype(jnp.int32))
    loss = _t6(ug, pg, ng).reshape(())
    return (loss, u, iv)


# speedup vs baseline: 17.3249x; 1.2426x over previous
import functools

import jax
import jax.numpy as jnp
from jax import lax
from jax.experimental import pallas as pl
from jax.experimental.pallas import tpu as pltpu
from jax.experimental.pallas import tpu_sc as plsc

NU = 30000
NI = 70000
D = 64
NNZ = 1000000
NB = 4096
NLAYERS = 3.0
DECAY = 1e-4

NC = 2
NS = 16
CH = 800
EPAD = 1024000
NPAD = EPAD - NNZ
NDUM = 1600
AU = NU + 2000
AI = NI + 2000
EW = EPAD // NS
NCH = EW // CH
EWD = EPAD // (NC * NS)
NCHD = EWD // CH

_mesh = plsc.VectorSubcoreMesh(core_axis_name="c", subcore_axis_name="s",
                               num_cores=NC, num_subcores=NS)
_sc_params = pltpu.CompilerParams(use_tc_tiling_on_sc=False)


def _sc_spmm_u_body(rows_s_ref, cols_g_ref, item_t_ref,
                    hu_ref, deg_ref,
                    cidx_a, cidx_b, gidx_a, gidx_b, ridx_a, ridx_b, didx,
                    rowsv_a, rowsv_b, onesv, zdeg,
                    acc, dacc, semga, semgb, semsc, semd):
    c = lax.axis_index("c")
    s = lax.axis_index("s")

    def _init(t, _):
        onesv[pl.ds(t * 16, 16)] = jnp.full((16,), 1.0, jnp.float32)
        return 0
    lax.fori_loop(0, CH // 16, _init, 0)

    def _zrow(r, _):
        rowsv_a[r, pl.ds(0, 16)] = jnp.zeros((16,), jnp.float32)
        rowsv_a[r, pl.ds(16, 16)] = jnp.zeros((16,), jnp.float32)
        return 0
    lax.fori_loop(0, CH, _zrow, 0)

    def _zdeg(t, _):
        zdeg[pl.ds(t * 16, 16)] = jnp.zeros((16,), jnp.float32)
        return 0
    lax.fori_loop(0, 2000 // 16, _zdeg, 0)

    pltpu.sync_copy(rowsv_a, acc.at[pl.ds(s * 2000, CH)])
    pltpu.sync_copy(rowsv_a, acc.at[pl.ds(s * 2000 + CH, CH)])
    pltpu.sync_copy(rowsv_a.at[pl.ds(0, 400)],
                    acc.at[pl.ds(s * 2000 + 2 * CH, 400)])
    pltpu.sync_copy(zdeg, dacc.at[pl.ds(s * 2000, 2000)])
    plsc.subcore_barrier()

    coff = c * NI

    def _pair(t, _):
        base_a = s * EW + (2 * t) * CH
        base_b = base_a + CH
        pltpu.sync_copy(cols_g_ref.at[pl.ds(base_a, CH)], cidx_a)
        pltpu.sync_copy(cols_g_ref.at[pl.ds(base_b, CH)], cidx_b)

        def _off(q, __):
            gidx_a[pl.ds(q * 16, 16)] = cidx_a[pl.ds(q * 16, 16)] + coff
            gidx_b[pl.ds(q * 16, 16)] = cidx_b[pl.ds(q * 16, 16)] + coff
            return 0
        lax.fori_loop(0, CH // 16, _off, 0)

        @pl.when(t > 0)
        def _drain():
            pltpu.make_async_copy(rowsv_a, acc.at[ridx_a], semsc).wait()
            pltpu.make_async_copy(rowsv_b, acc.at[ridx_b], semsc).wait()
            pltpu.make_async_copy(onesv, dacc.at[didx], semd).wait()

        ga = pltpu.async_copy(item_t_ref.at[gidx_a], rowsv_a, semga)
        gb = pltpu.async_copy(item_t_ref.at[gidx_b], rowsv_b, semgb)
        pltpu.sync_copy(rows_s_ref.at[pl.ds(base_a, CH)], ridx_a)
        pltpu.sync_copy(rows_s_ref.at[pl.ds(base_b, CH)], ridx_b)
        dbase = (c * NS + s) * EWD + t * CH
        pltpu.sync_copy(rows_s_ref.at[pl.ds(dbase, CH)], didx)
        ga.wait()
        pltpu.async_copy(rowsv_a, acc.at[ridx_a], semsc, add=True)
        gb.wait()
        pltpu.async_copy(rowsv_b, acc.at[ridx_b], semsc, add=True)
        pltpu.async_copy(onesv, dacc.at[didx], semd, add=True)
        return 0
    lax.fori_loop(0, NCH // 2, _pair, 0)

    pltpu.make_async_copy(rowsv_a, acc.at[ridx_a], semsc).wait()
    pltpu.make_async_copy(rowsv_b, acc.at[ridx_b], semsc).wait()
    pltpu.make_async_copy(onesv, dacc.at[didx], semd).wait()
    plsc.subcore_barrier()

    for off, sz in ((0, CH), (CH, CH), (2 * CH, 400)):
        pltpu.sync_copy(acc.at[pl.ds(s * 2000 + off, sz)],
                        rowsv_a.at[pl.ds(0, sz)])
        pltpu.sync_copy(rowsv_a.at[pl.ds(0, sz)],
                        hu_ref.at[pl.ds(c * AU + s * 2000 + off, sz)])
    pltpu.sync_copy(dacc.at[pl.ds(s * 2000, 2000)], zdeg)
    pltpu.sync_copy(zdeg, deg_ref.at[pl.ds(c * AU + s * 2000, 2000)])


_sc_spmm_u = functools.partial(
    pl.kernel,
    out_type=[jax.ShapeDtypeStruct((NC * AU, 32), jnp.float32),
              jax.ShapeDtypeStruct((NC * AU,), jnp.float32)],
    mesh=_mesh,
    scratch_types=[
        pltpu.VMEM((CH,), jnp.int32),
        pltpu.VMEM((CH,), jnp.int32),
        pltpu.VMEM((CH,), jnp.int32),
        pltpu.VMEM((CH,), jnp.int32),
        pltpu.VMEM((CH,), jnp.int32),
        pltpu.VMEM((CH,), jnp.int32),
        pltpu.VMEM((CH,), jnp.int32),
        pltpu.VMEM((CH, 32), jnp.float32),
        pltpu.VMEM((CH, 32), jnp.float32),
        pltpu.VMEM((CH,), jnp.float32),
        pltpu.VMEM((2000,), jnp.float32),
        pltpu.VMEM_SHARED((AU, 32), jnp.float32),
        pltpu.VMEM_SHARED((AU,), jnp.float32),
        pltpu.SemaphoreType.DMA,
        pltpu.SemaphoreType.DMA,
        pltpu.SemaphoreType.DMA,
        pltpu.SemaphoreType.DMA,
    ],
    compiler_params=_sc_params,
)(_sc_spmm_u_body)


def _sc_spmm_i_body(rows_g_ref, cols_s_ref, su_ref,
                    hi_ref,
                    cidx_a, cidx_b, gidx_a, gidx_b, ridx_a, ridx_b,
                    rowsv_a, rowsv_b,
                    acc, semga, semgb, semsc):
    c = lax.axis_index("c")
    s = lax.axis_index("s")
    for j in range(2):
        sl = c * 2 + j

        def _zrow(r, _):
            rowsv_a[r, pl.ds(0, 16)] = jnp.zeros((16,), jnp.float32)
            return 0
        lax.fori_loop(0, CH, _zrow, 0)

        for q in range(5):
            pltpu.sync_copy(rowsv_a, acc.at[pl.ds(s * 4500 + q * CH, CH)])
        pltpu.sync_copy(rowsv_a.at[pl.ds(0, 500)],
                        acc.at[pl.ds(s * 4500 + 5 * CH, 500)])
        plsc.subcore_barrier()
        soff = sl * NU

        def _pair(t, _):
            base_a = s * EW + (2 * t) * CH
            base_b = base_a + CH
            pltpu.sync_copy(rows_g_ref.at[pl.ds(base_a, CH)], ridx_a)
            pltpu.sync_copy(rows_g_ref.at[pl.ds(base_b, CH)], ridx_b)

            def _off(q, __):
                gidx_a[pl.ds(q * 16, 16)] = ridx_a[pl.ds(q * 16, 16)] + soff
                gidx_b[pl.ds(q * 16, 16)] = ridx_b[pl.ds(q * 16, 16)] + soff
                return 0
            lax.fori_loop(0, CH // 16, _off, 0)

            @pl.when(t > 0)
            def _drain():
                pltpu.make_async_copy(rowsv_a, acc.at[cidx_a], semsc).wait()
                pltpu.make_async_copy(rowsv_b, acc.at[cidx_b], semsc).wait()

            ga = pltpu.async_copy(su_ref.at[gidx_a], rowsv_a, semga)
            gb = pltpu.async_copy(su_ref.at[gidx_b], rowsv_b, semgb)
            pltpu.sync_copy(cols_s_ref.at[pl.ds(base_a, CH)], cidx_a)
            pltpu.sync_copy(cols_s_ref.at[pl.ds(base_b, CH)], cidx_b)
            ga.wait()
            pltpu.async_copy(rowsv_a, acc.at[cidx_a], semsc, add=True)
            gb.wait()
            pltpu.async_copy(rowsv_b, acc.at[cidx_b], semsc, add=True)
            return 0
        lax.fori_loop(0, NCH // 2, _pair, 0)

        pltpu.make_async_copy(rowsv_a, acc.at[cidx_a], semsc).wait()
        pltpu.make_async_copy(rowsv_b, acc.at[cidx_b], semsc).wait()
        plsc.subcore_barrier()

        for q in range(5):
            pltpu.sync_copy(acc.at[pl.ds(s * 4500 + q * CH, CH)], rowsv_a)
            pltpu.sync_copy(
                rowsv_a, hi_ref.at[pl.ds(sl * AI + s * 4500 + q * CH, CH)])
        pltpu.sync_copy(acc.at[pl.ds(s * 4500 + 5 * CH, 500)],
                        rowsv_a.at[pl.ds(0, 500)])
        pltpu.sync_copy(rowsv_a.at[pl.ds(0, 500)],
                        hi_ref.at[pl.ds(sl * AI + s * 4500 + 5 * CH, 500)])


_sc_spmm_i = functools.partial(
    pl.kernel,
    out_type=jax.ShapeDtypeStruct((4 * AI, 16), jnp.float32),
    mesh=_mesh,
    scratch_types=[
        pltpu.VMEM((CH,), jnp.int32),
        pltpu.VMEM((CH,), jnp.int32),
        pltpu.VMEM((CH,), jnp.int32),
        pltpu.VMEM((CH,), jnp.int32),
        pltpu.VMEM((CH,), jnp.int32),
        pltpu.VMEM((CH,), jnp.int32),
        pltpu.VMEM((CH, 16), jnp.float32),
        pltpu.VMEM((CH, 16), jnp.float32),
        pltpu.VMEM_SHARED((AI, 16), jnp.float32),
        pltpu.SemaphoreType.DMA,
        pltpu.SemaphoreType.DMA,
        pltpu.SemaphoreType.DMA,
    ],
    compiler_params=_sc_params,
)(_sc_spmm_i_body)


def _sc_gather_body(u_ref, i_ref, bu_ref, bp_ref, bn_ref,
                    ug_ref, pg_ref, ng_ref,
                    idxv, rowsv, sem):
    c = lax.axis_index("c")
    s = lax.axis_index("s")
    n = NB // (NC * NS)
    base = (s * NC + c) * n
    for src, idx_hbm, out in ((u_ref, bu_ref, ug_ref),
                              (i_ref, bp_ref, pg_ref),
                              (i_ref, bn_ref, ng_ref)):
        pltpu.sync_copy(idx_hbm.at[pl.ds(base, n)], idxv)
        pltpu.async_copy(src.at[idxv], rowsv, sem).wait()
        pltpu.sync_copy(rowsv, out.at[pl.ds(base, n)])


_sc_gather = functools.partial(
    pl.kernel,
    out_type=[jax.ShapeDtypeStruct((NB, D), jnp.float32)] * 3,
    mesh=_mesh,
    scratch_types=[
        pltpu.VMEM((NB // (NC * NS),), jnp.int32),
        pltpu.VMEM((NB // (NC * NS), D), jnp.float32),
        pltpu.SemaphoreType.DMA,
    ],
    compiler_params=_sc_params,
)(_sc_gather_body)


def _t1_body(re_ref, mask_ref, ent_ref, re2_ref, mask2_ref, ent2_ref, w2_ref,
             rfi_ref, rfu_ref):
    def _rel(r, m, e):
        x = r[...]
        x = jnp.exp(x - jnp.max(x, axis=1, keepdims=True))
        sm = x / jnp.sum(x, axis=1, keepdims=True)
        return jnp.dot(sm * m[...], e[...], preferred_element_type=jnp.float32)
    rfi_ref[...] = jnp.dot(_rel(re_ref, mask_ref, ent_ref), w2_ref[...],
                           preferred_element_type=jnp.float32)
    rfu_ref[...] = jnp.dot(_rel(re2_ref, mask2_ref, ent2_ref), w2_ref[...],
                           preferred_element_type=jnp.float32)


def _t1(re, mask, ent, re2, mask2, ent2, w2):
    return pl.pallas_call(
        _t1_body,
        out_shape=[jax.ShapeDtypeStruct((16, D), jnp.float32),
                   jax.ShapeDtypeStruct((8, D), jnp.float32)],
    )(re, mask, ent, re2, mask2, ent2, w2)


_BLK = 2000


def _t2_body(sn_ref, ir_ref, w1_ref, rfi_ref, out_ref):
    out_ref[0] = (
        jnp.dot(sn_ref[...], w1_ref[0], preferred_element_type=jnp.float32)
        + jnp.dot(ir_ref[...], rfi_ref[0], preferred_element_type=jnp.float32))


def _t2(struct_node_emb, ir, w1h, rfih):
    nb = NI // _BLK
    return pl.pallas_call(
        _t2_body,
        grid=(2, nb),
        in_specs=[
            pl.BlockSpec((_BLK, D), lambda h, i: (NU // _BLK + i, 0)),
            pl.BlockSpec((_BLK, 16), lambda h, i: (i, 0)),
            pl.BlockSpec((1, D, 32), lambda h, i: (h, 0, 0)),
            pl.BlockSpec((1, 16, 32), lambda h, i: (h, 0, 0)),
        ],
        out_specs=pl.BlockSpec((1, _BLK, 32), lambda h, i: (h, i, 0)),
        out_shape=jax.ShapeDtypeStruct((2, NI, 32), jnp.float32),
    )(struct_node_emb, ir, w1h, rfih)


def _t3_body(sn_ref, ur_ref, w1_ref, rfu_ref, out_ref):
    out_ref[...] = (
        jnp.dot(sn_ref[...], w1_ref[...], preferred_element_type=jnp.float32)
        + jnp.dot(ur_ref[...], rfu_ref[...], preferred_element_type=jnp.float32))


def _t3(struct_node_emb, ur, w1, rfu):
    nb = NU // _BLK
    return pl.pallas_call(
        _t3_body,
        grid=(nb,),
        in_specs=[
            pl.BlockSpec((_BLK, D), lambda i: (i, 0)),
            pl.BlockSpec((_BLK, 8), lambda i: (i, 0)),
            pl.BlockSpec((D, D), lambda i: (0, 0)),
            pl.BlockSpec((8, D), lambda i: (0, 0)),
        ],
        out_specs=pl.BlockSpec((_BLK, D), lambda i: (i, 0)),
        out_shape=jax.ShapeDtypeStruct((NU, D), jnp.float32),
    )(struct_node_emb, ur, w1, rfu)


def _t4_body(uc_ref, d0_ref, d1_ref, o0, o1, o2, o3):
    deg = d0_ref[...] + d1_ref[...]
    dinv = jnp.where(deg > 0, 1.0 / deg, 0.0)
    x = uc_ref[...] * dinv
    o0[...] = x[:, 0:16]
    o1[...] = x[:, 16:32]
    o2[...] = x[:, 32:48]
    o3[...] = x[:, 48:64]


def _t4(user_comb, deg0, deg1):
    nb = NU // _BLK
    return pl.pallas_call(
        _t4_body,
        grid=(nb,),
        in_specs=[
            pl.BlockSpec((_BLK, D), lambda i: (i, 0)),
            pl.BlockSpec((_BLK, 1), lambda i: (i, 0)),
            pl.BlockSpec((_BLK, 1), lambda i: (i, 0)),
        ],
        out_specs=[pl.BlockSpec((_BLK, 16), lambda i: (i, 0))] * 4,
        out_shape=[jax.ShapeDtypeStruct((NU, 16), jnp.float32)] * 4,
    )(user_comb, deg0, deg1)


def _finalize(x):
    x = jnp.where(x > 0, x, 0.2 * x)
    n = jnp.sqrt(jnp.sum(x * x, axis=1, keepdims=True))
    return NLAYERS * x / jnp.maximum(n, 1e-12)


def _t5u_body(a_ref, b_ref, d0_ref, d1_ref, bias_ref, out_ref):
    deg = d0_ref[...] + d1_ref[...]
    dinv = jnp.where(deg > 0, 1.0 / deg, 0.0)
    h = jnp.concatenate([a_ref[...], b_ref[...]], axis=1) * dinv + bias_ref[...]
    out_ref[...] = _finalize(h)


def _t5u(hu_a, hu_b, deg0, deg1, bias):
    nb = NU // _BLK
    return pl.pallas_call(
        _t5u_body,
        grid=(nb,),
        in_specs=[
            pl.BlockSpec((_BLK, 32), lambda i: (i, 0)),
            pl.BlockSpec((_BLK, 32), lambda i: (i, 0)),
            pl.BlockSpec((_BLK, 1), lambda i: (i, 0)),
            pl.BlockSpec((_BLK, 1), lambda i: (i, 0)),
            pl.BlockSpec((_BLK, D), lambda i: (i, 0)),
        ],
        out_specs=pl.BlockSpec((_BLK, D), lambda i: (i, 0)),
        out_shape=jax.ShapeDtypeStruct((NU, D), jnp.float32),
    )(hu_a, hu_b, deg0, deg1, bias)


def _t5i_body(h0_ref, h1_ref, h2_ref, h3_ref, bias_ref, out_ref):
    h = jnp.concatenate(
        [h0_ref[...], h1_ref[...], h2_ref[...], h3_ref[...]], axis=1)
    out_ref[...] = _finalize(h + bias_ref[...])


def _t5i(h0, h1, h2, h3, bias):
    nb = NI // _BLK
    return pl.pallas_call(
        _t5i_body,
        grid=(nb,),
        in_specs=[pl.BlockSpec((_BLK, 16), lambda i: (i, 0))] * 4 + [
            pl.BlockSpec((_BLK, D), lambda i: (NU // _BLK + i, 0)),
        ],
        out_specs=pl.BlockSpec((_BLK, D), lambda i: (i, 0)),
        out_shape=jax.ShapeDtypeStruct((NI, D), jnp.float32),
    )(h0, h1, h2, h3, bias)


def _t6_body(ug_ref, pg_ref, ng_ref, out_ref):
    ug, pg, ng = ug_ref[...], pg_ref[...], ng_ref[...]
    ps = jnp.sum(ug * pg, axis=1, keepdims=True)
    ns = jnp.sum(ug * ng, axis=1, keepdims=True)
    x = ps - ns
    ls = jnp.minimum(x, 0.0) - jnp.log(1.0 + jnp.exp(-jnp.abs(x)))
    mf = -jnp.sum(ls) / NB
    reg = (jnp.sum(ug * ug) + jnp.sum(pg * pg) + jnp.sum(ng * ng)) * 0.5
    out_ref[...] = jnp.reshape(mf + DECAY * reg / NB, (1, 1))


def _t6(ug, pg, ng):
    return pl.pallas_call(
        _t6_body,
        out_shape=jax.ShapeDtypeStruct((1, 1), jnp.float32),
    )(ug, pg, ng)


def kernel(struct_node_emb, train_weight, train_weight_2, bias, re, entity_emb,
           ir, re_2, entity_emb_2, ur, mask, mask_2, rows, cols,
           batch_users, batch_pos, batch_neg):
    rows = rows.astype(jnp.int32)
    cols = cols.astype(jnp.int32)
    pad = jnp.arange(NPAD, dtype=jnp.int32)
    rows_s = jnp.concatenate([rows, NU + pad % NDUM])
    rows_g = jnp.concatenate([rows, pad % 16])
    cols_s = jnp.concatenate([cols, NI + pad % NDUM])
    cols_g = jnp.concatenate([cols, pad % 16])

    rfi, rfu = _t1(re, mask, entity_emb, re_2, mask_2, entity_emb_2,
                   train_weight_2)
    w1h = jnp.stack([train_weight[:, :32], train_weight[:, 32:]])
    rfih = jnp.stack([rfi[:, :32], rfi[:, 32:]])
    item_t = _t2(struct_node_emb, ir, w1h, rfih).reshape(2 * NI, 32)
    user_comb = _t3(struct_node_emb, ur, train_weight, rfu)

    hu_flat, deg_flat = _sc_spmm_u(rows_s, cols_g, item_t)

    deg0 = deg_flat[:NU].reshape(NU, 1)
    deg1 = deg_flat[AU:AU + NU].reshape(NU, 1)
    su = jnp.concatenate(_t4(user_comb, deg0, deg1), axis=0)

    hi_flat = _sc_spmm_i(rows_g, cols_s, su)

    u = _t5u(hu_flat[:NU], hu_flat[AU:AU + NU], deg0, deg1, bias)
    iv = _t5i(hi_flat[:NI], hi_flat[AI:AI + NI], hi_flat[2 * AI:2 * AI + NI],
              hi_flat[3 * AI:3 * AI + NI], bias)

    ug, pg, ng = _sc_gather(u, iv, batch_users.astype(jnp.int32),
                            batch_pos.astype(jnp.int32),
                            batch_neg.astype(jnp.int32))
    loss = _t6(ug, pg, ng).reshape(())
    return (loss, u, iv)


# speedup vs baseline: 17.8035x; 1.0276x over previous
import functools

import jax
import jax.numpy as jnp
from jax import lax
from jax.experimental import pallas as pl
from jax.experimental.pallas import tpu as pltpu
from jax.experimental.pallas import tpu_sc as plsc

NU = 30000
NI = 70000
D = 64
NNZ = 1000000
NB = 4096
NLAYERS = 3.0
DECAY = 1e-4

NC = 2
NS = 16
CH = 800
EPAD = 1024000
NPAD = EPAD - NNZ
NDUMU = 704
NDUMI = 1600
AU = NU + 720
AI = NI + 2000
SA = AU // NS
EW = EPAD // NS
NCH = EW // CH
GA = 4
GB = 10
TA = NCH // GA
TB = NCH // GB

_mesh = plsc.VectorSubcoreMesh(core_axis_name="c", subcore_axis_name="s",
                               num_cores=NC, num_subcores=NS)
_sc_params = pltpu.CompilerParams(use_tc_tiling_on_sc=False)


def _sc_spmm_u_body(rows_s_ref, cols_g_ref, item_t_ref,
                    hu_ref, deg_ref,
                    cidx_blk, ridx_blk, gidx_a, gidx_b, onesv, zdeg,
                    rowsv_a, rowsv_b,
                    acc, dacc, semga, semgb, semsa, semsb, semd):
    c = lax.axis_index("c")
    s = lax.axis_index("s")
    gidx = (gidx_a, gidx_b)
    rowsv = (rowsv_a, rowsv_b)
    gsem = (semga, semgb)
    ssem = (semsa, semsb)

    def _init(t, _):
        onesv[pl.ds(t * 16, 16)] = jnp.full((16,), 1.0, jnp.float32)
        return 0
    lax.fori_loop(0, CH // 16, _init, 0)

    def _zrow(r, _):
        rowsv_a[r, pl.ds(0, 16)] = jnp.zeros((16,), jnp.float32)
        rowsv_a[r, pl.ds(16, 16)] = jnp.zeros((16,), jnp.float32)
        return 0
    lax.fori_loop(0, CH, _zrow, 0)

    def _zdeg(t, _):
        zdeg[pl.ds(t * 16, 16)] = jnp.zeros((16,), jnp.float32)
        return 0
    lax.fori_loop(0, SA // 16, _zdeg, 0)

    pltpu.sync_copy(rowsv_a, acc.at[pl.ds(s * SA, CH)])
    pltpu.sync_copy(rowsv_a, acc.at[pl.ds(s * SA + CH, CH)])
    pltpu.sync_copy(rowsv_a.at[pl.ds(0, SA - 2 * CH)],
                    acc.at[pl.ds(s * SA + 2 * CH, SA - 2 * CH)])
    pltpu.sync_copy(zdeg, dacc.at[pl.ds(s * SA, SA)])
    plsc.subcore_barrier()

    coff = c * NI

    def _outer(t, _):
        @pl.when(t > 0)
        def _dd():
            pltpu.make_async_copy(onesv, dacc.at[ridx_blk.at[0]], semd).wait()
            pltpu.make_async_copy(onesv, dacc.at[ridx_blk.at[0]], semd).wait()
        blk = s * (EW // CH) + t * GA
        pltpu.sync_copy(cols_g_ref.at[pl.ds(blk, GA)], cidx_blk)
        pltpu.sync_copy(rows_s_ref.at[pl.ds(blk, GA)], ridx_blk)

        @pl.when(c == 0)
        def _d0():
            pltpu.async_copy(onesv, dacc.at[ridx_blk.at[0]], semd, add=True)
            pltpu.async_copy(onesv, dacc.at[ridx_blk.at[1]], semd, add=True)

        @pl.when(c == 1)
        def _d1():
            pltpu.async_copy(onesv, dacc.at[ridx_blk.at[2]], semd, add=True)
            pltpu.async_copy(onesv, dacc.at[ridx_blk.at[3]], semd, add=True)

        def _gidx(q, dst):
            def _off(w, __):
                dst[pl.ds(w * 16, 16)] = (
                    cidx_blk[q, pl.ds(w * 16, 16)] + coff)
                return 0
            lax.fori_loop(0, CH // 16, _off, 0)

        gd = [None, None]
        sd = [None, None]
        _gidx(0, gidx[0])
        gd[0] = pltpu.async_copy(item_t_ref.at[gidx[0]], rowsv[0], gsem[0])
        for q in range(GA):
            p = q & 1
            pn = 1 - p
            if q + 1 < GA:
                if sd[pn] is not None:
                    sd[pn].wait()
                    sd[pn] = None
                _gidx(q + 1, gidx[pn])
                gd[pn] = pltpu.async_copy(
                    item_t_ref.at[gidx[pn]], rowsv[pn], gsem[pn])
            gd[p].wait()
            sd[p] = pltpu.async_copy(
                rowsv[p], acc.at[ridx_blk.at[q]], ssem[p], add=True)
        for b in range(2):
            if sd[b] is not None:
                sd[b].wait()
        return 0
    lax.fori_loop(0, TA, _outer, 0)

    pltpu.make_async_copy(onesv, dacc.at[ridx_blk.at[0]], semd).wait()
    pltpu.make_async_copy(onesv, dacc.at[ridx_blk.at[0]], semd).wait()
    plsc.subcore_barrier()

    for off, sz in ((0, CH), (CH, CH), (2 * CH, SA - 2 * CH)):
        pltpu.sync_copy(acc.at[pl.ds(s * SA + off, sz)],
                        rowsv_a.at[pl.ds(0, sz)])
        pltpu.sync_copy(rowsv_a.at[pl.ds(0, sz)],
                        hu_ref.at[pl.ds(c * AU + s * SA + off, sz)])
    pltpu.sync_copy(dacc.at[pl.ds(s * SA, SA)], zdeg)
    pltpu.sync_copy(zdeg, deg_ref.at[pl.ds(c * AU + s * SA, SA)])


_sc_spmm_u = functools.partial(
    pl.kernel,
    out_type=[jax.ShapeDtypeStruct((NC * AU, 32), jnp.float32),
              jax.ShapeDtypeStruct((NC * AU,), jnp.float32)],
    mesh=_mesh,
    scratch_types=[
        pltpu.VMEM((GA, CH), jnp.int32),
        pltpu.VMEM((GA, CH), jnp.int32),
        pltpu.VMEM((CH,), jnp.int32),
        pltpu.VMEM((CH,), jnp.int32),
        pltpu.VMEM((CH,), jnp.float32),
        pltpu.VMEM((SA,), jnp.float32),
        pltpu.VMEM((CH, 32), jnp.float32),
        pltpu.VMEM((CH, 32), jnp.float32),
        pltpu.VMEM_SHARED((AU, 32), jnp.float32),
        pltpu.VMEM_SHARED((AU,), jnp.float32),
        pltpu.SemaphoreType.DMA,
        pltpu.SemaphoreType.DMA,
        pltpu.SemaphoreType.DMA,
        pltpu.SemaphoreType.DMA,
        pltpu.SemaphoreType.DMA,
    ],
    compiler_params=_sc_params,
)(_sc_spmm_u_body)


def _sc_spmm_i_body(rows_g_ref, cols_s_ref, su_ref,
                    hi_ref,
                    ridx_blk, cidx_blk, gidx_a, gidx_b,
                    rowsv_a, rowsv_b,
                    acc, semga, semgb, semsa, semsb):
    c = lax.axis_index("c")
    s = lax.axis_index("s")
    gidx = (gidx_a, gidx_b)
    rowsv = (rowsv_a, rowsv_b)
    gsem = (semga, semgb)
    ssem = (semsa, semsb)
    for j in range(2):
        sl = c * 2 + j

        def _zrow(r, _):
            rowsv_a[r, pl.ds(0, 16)] = jnp.zeros((16,), jnp.float32)
            return 0
        lax.fori_loop(0, CH, _zrow, 0)

        for q in range(5):
            pltpu.sync_copy(rowsv_a, acc.at[pl.ds(s * 4500 + q * CH, CH)])
        pltpu.sync_copy(rowsv_a.at[pl.ds(0, 500)],
                        acc.at[pl.ds(s * 4500 + 5 * CH, 500)])
        plsc.subcore_barrier()
        soff = sl * NU

        def _outer(t, _):
            blk = s * (EW // CH) + t * GB
            pltpu.sync_copy(rows_g_ref.at[pl.ds(blk, GB)], ridx_blk)
            pltpu.sync_copy(cols_s_ref.at[pl.ds(blk, GB)], cidx_blk)

            def _gidx(q, dst):
                def _off(w, __):
                    dst[pl.ds(w * 16, 16)] = (
                        ridx_blk[q, pl.ds(w * 16, 16)] + soff)
                    return 0
                lax.fori_loop(0, CH // 16, _off, 0)

            gd = [None, None]
            sd = [None, None]
            _gidx(0, gidx[0])
            gd[0] = pltpu.async_copy(su_ref.at[gidx[0]], rowsv[0], gsem[0])
            for q in range(GB):
                p = q & 1
                pn = 1 - p
                if q + 1 < GB:
                    if sd[pn] is not None:
                        sd[pn].wait()
                        sd[pn] = None
                    _gidx(q + 1, gidx[pn])
                    gd[pn] = pltpu.async_copy(
                        su_ref.at[gidx[pn]], rowsv[pn], gsem[pn])
                gd[p].wait()
                sd[p] = pltpu.async_copy(
                    rowsv[p], acc.at[cidx_blk.at[q]], ssem[p], add=True)
            for b in range(2):
                if sd[b] is not None:
                    sd[b].wait()
            return 0
        lax.fori_loop(0, TB, _outer, 0)
        plsc.subcore_barrier()

        for q in range(5):
            pltpu.sync_copy(acc.at[pl.ds(s * 4500 + q * CH, CH)], rowsv_a)
            pltpu.sync_copy(
                rowsv_a, hi_ref.at[pl.ds(sl * AI + s * 4500 + q * CH, CH)])
        pltpu.sync_copy(acc.at[pl.ds(s * 4500 + 5 * CH, 500)],
                        rowsv_a.at[pl.ds(0, 500)])
        pltpu.sync_copy(rowsv_a.at[pl.ds(0, 500)],
                        hi_ref.at[pl.ds(sl * AI + s * 4500 + 5 * CH, 500)])


_sc_spmm_i = functools.partial(
    pl.kernel,
    out_type=jax.ShapeDtypeStruct((4 * AI, 16), jnp.float32),
    mesh=_mesh,
    scratch_types=[
        pltpu.VMEM((GB, CH), jnp.int32),
        pltpu.VMEM((GB, CH), jnp.int32),
        pltpu.VMEM((CH,), jnp.int32),
        pltpu.VMEM((CH,), jnp.int32),
        pltpu.VMEM((CH, 16), jnp.float32),
        pltpu.VMEM((CH, 16), jnp.float32),
        pltpu.VMEM_SHARED((AI, 16), jnp.float32),
        pltpu.SemaphoreType.DMA,
        pltpu.SemaphoreType.DMA,
        pltpu.SemaphoreType.DMA,
        pltpu.SemaphoreType.DMA,
    ],
    compiler_params=_sc_params,
)(_sc_spmm_i_body)


def _sc_gather_body(u_ref, i_ref, bu_ref, bp_ref, bn_ref,
                    ug_ref, pg_ref, ng_ref,
                    idxv, rowsv, sem):
    c = lax.axis_index("c")
    s = lax.axis_index("s")
    n = NB // (NC * NS)
    base = (s * NC + c) * n
    for src, idx_hbm, out in ((u_ref, bu_ref, ug_ref),
                              (i_ref, bp_ref, pg_ref),
                              (i_ref, bn_ref, ng_ref)):
        pltpu.sync_copy(idx_hbm.at[pl.ds(base, n)], idxv)
        pltpu.async_copy(src.at[idxv], rowsv, sem).wait()
        pltpu.sync_copy(rowsv, out.at[pl.ds(base, n)])


_sc_gather = functools.partial(
    pl.kernel,
    out_type=[jax.ShapeDtypeStruct((NB, D), jnp.float32)] * 3,
    mesh=_mesh,
    scratch_types=[
        pltpu.VMEM((NB // (NC * NS),), jnp.int32),
        pltpu.VMEM((NB // (NC * NS), D), jnp.float32),
        pltpu.SemaphoreType.DMA,
    ],
    compiler_params=_sc_params,
)(_sc_gather_body)


def _t1_body(re_ref, mask_ref, ent_ref, re2_ref, mask2_ref, ent2_ref, w2_ref,
             rfi_ref, rfu_ref):
    def _rel(r, m, e):
        x = r[...]
        x = jnp.exp(x - jnp.max(x, axis=1, keepdims=True))
        sm = x / jnp.sum(x, axis=1, keepdims=True)
        return jnp.dot(sm * m[...], e[...], preferred_element_type=jnp.float32)
    rfi_ref[...] = jnp.dot(_rel(re_ref, mask_ref, ent_ref), w2_ref[...],
                           preferred_element_type=jnp.float32)
    rfu_ref[...] = jnp.dot(_rel(re2_ref, mask2_ref, ent2_ref), w2_ref[...],
                           preferred_element_type=jnp.float32)


def _t1(re, mask, ent, re2, mask2, ent2, w2):
    return pl.pallas_call(
        _t1_body,
        out_shape=[jax.ShapeDtypeStruct((16, D), jnp.float32),
                   jax.ShapeDtypeStruct((8, D), jnp.float32)],
    )(re, mask, ent, re2, mask2, ent2, w2)


_BLK = 2000


def _t2_body(sn_ref, ir_ref, w1_ref, rfi_ref, out_ref):
    out_ref[0] = (
        jnp.dot(sn_ref[...], w1_ref[0], preferred_element_type=jnp.float32)
        + jnp.dot(ir_ref[...], rfi_ref[0], preferred_element_type=jnp.float32))


def _t2(struct_node_emb, ir, w1h, rfih):
    nb = NI // _BLK
    return pl.pallas_call(
        _t2_body,
        grid=(2, nb),
        in_specs=[
            pl.BlockSpec((_BLK, D), lambda h, i: (NU // _BLK + i, 0)),
            pl.BlockSpec((_BLK, 16), lambda h, i: (i, 0)),
            pl.BlockSpec((1, D, 32), lambda h, i: (h, 0, 0)),
            pl.BlockSpec((1, 16, 32), lambda h, i: (h, 0, 0)),
        ],
        out_specs=pl.BlockSpec((1, _BLK, 32), lambda h, i: (h, i, 0)),
        out_shape=jax.ShapeDtypeStruct((2, NI, 32), jnp.float32),
    )(struct_node_emb, ir, w1h, rfih)


def _t3_body(sn_ref, ur_ref, w1_ref, rfu_ref, out_ref):
    out_ref[...] = (
        jnp.dot(sn_ref[...], w1_ref[...], preferred_element_type=jnp.float32)
        + jnp.dot(ur_ref[...], rfu_ref[...], preferred_element_type=jnp.float32))


def _t3(struct_node_emb, ur, w1, rfu):
    nb = NU // _BLK
    return pl.pallas_call(
        _t3_body,
        grid=(nb,),
        in_specs=[
            pl.BlockSpec((_BLK, D), lambda i: (i, 0)),
            pl.BlockSpec((_BLK, 8), lambda i: (i, 0)),
            pl.BlockSpec((D, D), lambda i: (0, 0)),
            pl.BlockSpec((8, D), lambda i: (0, 0)),
        ],
        out_specs=pl.BlockSpec((_BLK, D), lambda i: (i, 0)),
        out_shape=jax.ShapeDtypeStruct((NU, D), jnp.float32),
    )(struct_node_emb, ur, w1, rfu)


def _t4_body(uc_ref, d0_ref, d1_ref, o0, o1, o2, o3):
    deg = d0_ref[...] + d1_ref[...]
    dinv = jnp.where(deg > 0, 1.0 / deg, 0.0)
    x = uc_ref[...] * dinv
    o0[...] = x[:, 0:16]
    o1[...] = x[:, 16:32]
    o2[...] = x[:, 32:48]
    o3[...] = x[:, 48:64]


def _t4(user_comb, deg0, deg1):
    nb = NU // _BLK
    return pl.pallas_call(
        _t4_body,
        grid=(nb,),
        in_specs=[
            pl.BlockSpec((_BLK, D), lambda i: (i, 0)),
            pl.BlockSpec((_BLK, 1), lambda i: (i, 0)),
            pl.BlockSpec((_BLK, 1), lambda i: (i, 0)),
        ],
        out_specs=[pl.BlockSpec((_BLK, 16), lambda i: (i, 0))] * 4,
        out_shape=[jax.ShapeDtypeStruct((NU, 16), jnp.float32)] * 4,
    )(user_comb, deg0, deg1)


def _finalize(x):
    x = jnp.where(x > 0, x, 0.2 * x)
    n = jnp.sqrt(jnp.sum(x * x, axis=1, keepdims=True))
    return NLAYERS * x / jnp.maximum(n, 1e-12)


def _t5u_body(a_ref, b_ref, d0_ref, d1_ref, bias_ref, out_ref):
    deg = d0_ref[...] + d1_ref[...]
    dinv = jnp.where(deg > 0, 1.0 / deg, 0.0)
    h = jnp.concatenate([a_ref[...], b_ref[...]], axis=1) * dinv + bias_ref[...]
    out_ref[...] = _finalize(h)


def _t5u(hu_a, hu_b, deg0, deg1, bias):
    nb = NU // _BLK
    return pl.pallas_call(
        _t5u_body,
        grid=(nb,),
        in_specs=[
            pl.BlockSpec((_BLK, 32), lambda i: (i, 0)),
            pl.BlockSpec((_BLK, 32), lambda i: (i, 0)),
            pl.BlockSpec((_BLK, 1), lambda i: (i, 0)),
            pl.BlockSpec((_BLK, 1), lambda i: (i, 0)),
            pl.BlockSpec((_BLK, D), lambda i: (i, 0)),
        ],
        out_specs=pl.BlockSpec((_BLK, D), lambda i: (i, 0)),
        out_shape=jax.ShapeDtypeStruct((NU, D), jnp.float32),
    )(hu_a, hu_b, deg0, deg1, bias)


def _t5i_body(h0_ref, h1_ref, h2_ref, h3_ref, bias_ref, out_ref):
    h = jnp.concatenate(
        [h0_ref[...], h1_ref[...], h2_ref[...], h3_ref[...]], axis=1)
    out_ref[...] = _finalize(h + bias_ref[...])


def _t5i(h0, h1, h2, h3, bias):
    nb = NI // _BLK
    return pl.pallas_call(
        _t5i_body,
        grid=(nb,),
        in_specs=[pl.BlockSpec((_BLK, 16), lambda i: (i, 0))] * 4 + [
            pl.BlockSpec((_BLK, D), lambda i: (NU // _BLK + i, 0)),
        ],
        out_specs=pl.BlockSpec((_BLK, D), lambda i: (i, 0)),
        out_shape=jax.ShapeDtypeStruct((NI, D), jnp.float32),
    )(h0, h1, h2, h3, bias)


def _t6_body(ug_ref, pg_ref, ng_ref, out_ref):
    ug, pg, ng = ug_ref[...], pg_ref[...], ng_ref[...]
    ps = jnp.sum(ug * pg, axis=1, keepdims=True)
    ns = jnp.sum(ug * ng, axis=1, keepdims=True)
    x = ps - ns
    ls = jnp.minimum(x, 0.0) - jnp.log(1.0 + jnp.exp(-jnp.abs(x)))
    mf = -jnp.sum(ls) / NB
    reg = (jnp.sum(ug * ug) + jnp.sum(pg * pg) + jnp.sum(ng * ng)) * 0.5
    out_ref[...] = jnp.reshape(mf + DECAY * reg / NB, (1, 1))


def _t6(ug, pg, ng):
    return pl.pallas_call(
        _t6_body,
        out_shape=jax.ShapeDtypeStruct((1, 1), jnp.float32),
    )(ug, pg, ng)


def kernel(struct_node_emb, train_weight, train_weight_2, bias, re, entity_emb,
           ir, re_2, entity_emb_2, ur, mask, mask_2, rows, cols,
           batch_users, batch_pos, batch_neg):
    rows = rows.astype(jnp.int32)
    cols = cols.astype(jnp.int32)
    pad = jnp.arange(NPAD, dtype=jnp.int32)
    blk2 = (EPAD // CH, CH)
    rows_s = jnp.concatenate([rows, NU + pad % NDUMU]).reshape(blk2)
    rows_g = jnp.concatenate([rows, pad % 16]).reshape(blk2)
    cols_s = jnp.concatenate([cols, NI + pad % NDUMI]).reshape(blk2)
    cols_g = jnp.concatenate([cols, pad % 16]).reshape(blk2)

    rfi, rfu = _t1(re, mask, entity_emb, re_2, mask_2, entity_emb_2,
                   train_weight_2)
    w1h = jnp.stack([train_weight[:, :32], train_weight[:, 32:]])
    rfih = jnp.stack([rfi[:, :32], rfi[:, 32:]])
    item_t = _t2(struct_node_emb, ir, w1h, rfih).reshape(2 * NI, 32)
    user_comb = _t3(struct_node_emb, ur, train_weight, rfu)

    hu_flat, deg_flat = _sc_spmm_u(rows_s, cols_g, item_t)

    deg0 = deg_flat[:NU].reshape(NU, 1)
    deg1 = deg_flat[AU:AU + NU].reshape(NU, 1)
    su = jnp.concatenate(_t4(user_comb, deg0, deg1), axis=0)

    hi_flat = _sc_spmm_i(rows_g, cols_s, su)

    u = _t5u(hu_flat[:NU], hu_flat[AU:AU + NU], deg0, deg1, bias)
    iv = _t5i(hi_flat[:NI], hi_flat[AI:AI + NI], hi_flat[2 * AI:2 * AI + NI],
              hi_flat[3 * AI:3 * AI + NI], bias)

    ug, pg, ng = _sc_gather(u, iv, batch_users.astype(jnp.int32),
                            batch_pos.astype(jnp.int32),
                            batch_neg.astype(jnp.int32))
    loss = _t6(ug, pg, ng).reshape(())
    return (loss, u, iv)


# speedup vs baseline: 27.2530x; 1.5308x over previous
import functools

import jax
import jax.numpy as jnp
from jax import lax
from jax.experimental import pallas as pl
from jax.experimental.pallas import tpu as pltpu
from jax.experimental.pallas import tpu_sc as plsc

NU = 30000
NI = 70000
D = 64
NNZ = 1000000
NB = 4096
NLAYERS = 3.0
DECAY = 1e-4

NC = 2
NS = 16
CH = 800
EPAD = 1024000
NPAD = EPAD - NNZ
NDUMU = 704
NDUMI = 1600
AU = NU + 720
AI = NI + 2000
SA = AU // NS
EW = EPAD // NS
NCH = EW // CH
GA = 4
GB = 10
TA = NCH // GA
TB = NCH // GB

_mesh = plsc.VectorSubcoreMesh(core_axis_name="c", subcore_axis_name="s",
                               num_cores=NC, num_subcores=NS)
_sc_params = pltpu.CompilerParams(use_tc_tiling_on_sc=False)


def _sc_spmm_u_body(rows_s_ref, cols_g_ref, item_t_ref,
                    hu_ref, deg_ref,
                    cidx_blk, ridx_blk, gidx_a, gidx_b, onesv, zdeg,
                    rowsv_a, rowsv_b,
                    acc, dacc, semga, semgb, semsa, semsb, semd):
    c = lax.axis_index("c")
    s = lax.axis_index("s")
    gidx = (gidx_a, gidx_b)
    rowsv = (rowsv_a, rowsv_b)
    gsem = (semga, semgb)
    ssem = (semsa, semsb)

    def _init(t, _):
        onesv[pl.ds(t * 16, 16)] = jnp.full((16,), 1.0, jnp.float32)
        return 0
    lax.fori_loop(0, CH // 16, _init, 0)

    def _zrow(r, _):
        rowsv_a[r, pl.ds(0, 16)] = jnp.zeros((16,), jnp.float32)
        rowsv_a[r, pl.ds(16, 16)] = jnp.zeros((16,), jnp.float32)
        return 0
    lax.fori_loop(0, CH, _zrow, 0)

    def _zdeg(t, _):
        zdeg[pl.ds(t * 16, 16)] = jnp.zeros((16,), jnp.float32)
        return 0
    lax.fori_loop(0, SA // 16, _zdeg, 0)

    pltpu.sync_copy(rowsv_a, acc.at[pl.ds(s * SA, CH)])
    pltpu.sync_copy(rowsv_a, acc.at[pl.ds(s * SA + CH, CH)])
    pltpu.sync_copy(rowsv_a.at[pl.ds(0, SA - 2 * CH)],
                    acc.at[pl.ds(s * SA + 2 * CH, SA - 2 * CH)])
    pltpu.sync_copy(zdeg, dacc.at[pl.ds(s * SA, SA)])
    plsc.subcore_barrier()

    coff = c

    def _outer(t, _):
        @pl.when(t > 0)
        def _dd():
            pltpu.make_async_copy(onesv, dacc.at[ridx_blk.at[0]], semd).wait()
            pltpu.make_async_copy(onesv, dacc.at[ridx_blk.at[0]], semd).wait()
        blk = s * (EW // CH) + t * GA
        pltpu.sync_copy(cols_g_ref.at[pl.ds(blk, GA)], cidx_blk)
        pltpu.sync_copy(rows_s_ref.at[pl.ds(blk, GA)], ridx_blk)

        @pl.when(c == 0)
        def _d0():
            pltpu.async_copy(onesv, dacc.at[ridx_blk.at[0]], semd, add=True)
            pltpu.async_copy(onesv, dacc.at[ridx_blk.at[1]], semd, add=True)

        @pl.when(c == 1)
        def _d1():
            pltpu.async_copy(onesv, dacc.at[ridx_blk.at[2]], semd, add=True)
            pltpu.async_copy(onesv, dacc.at[ridx_blk.at[3]], semd, add=True)

        def _gidx(q, dst):
            def _off(w, __):
                dst[pl.ds(w * 16, 16)] = (
                    cidx_blk[q, pl.ds(w * 16, 16)] * 2 + coff)
                return 0
            lax.fori_loop(0, CH // 16, _off, 0)

        gd = [None, None]
        sd = [None, None]
        _gidx(0, gidx[0])
        gd[0] = pltpu.async_copy(item_t_ref.at[gidx[0]], rowsv[0], gsem[0])
        for q in range(GA):
            p = q & 1
            pn = 1 - p
            if q + 1 < GA:
                if sd[pn] is not None:
                    sd[pn].wait()
                    sd[pn] = None
                _gidx(q + 1, gidx[pn])
                gd[pn] = pltpu.async_copy(
                    item_t_ref.at[gidx[pn]], rowsv[pn], gsem[pn])
            gd[p].wait()
            sd[p] = pltpu.async_copy(
                rowsv[p], acc.at[ridx_blk.at[q]], ssem[p], add=True)
        for b in range(2):
            if sd[b] is not None:
                sd[b].wait()
        return 0
    lax.fori_loop(0, TA, _outer, 0)

    pltpu.make_async_copy(onesv, dacc.at[ridx_blk.at[0]], semd).wait()
    pltpu.make_async_copy(onesv, dacc.at[ridx_blk.at[0]], semd).wait()
    plsc.subcore_barrier()

    def _wb(colo):
        for off, sz in ((0, CH), (CH, CH), (2 * CH, 275)):
            pltpu.sync_copy(acc.at[pl.ds(s * 1875 + off, sz)],
                            rowsv_a.at[pl.ds(0, sz)])
            pltpu.sync_copy(
                rowsv_a.at[pl.ds(0, sz)],
                hu_ref.at[pl.ds(s * 1875 + off, sz), pl.ds(colo, 32)])

    @pl.when(c == 0)
    def _wb0():
        _wb(0)

    @pl.when(c == 1)
    def _wb1():
        _wb(32)

    pltpu.sync_copy(dacc.at[pl.ds(s * SA, SA)], zdeg)
    pltpu.sync_copy(zdeg, deg_ref.at[pl.ds(c * AU + s * SA, SA)])


_sc_spmm_u = functools.partial(
    pl.kernel,
    out_type=[jax.ShapeDtypeStruct((NU, D), jnp.float32),
              jax.ShapeDtypeStruct((NC * AU,), jnp.float32)],
    mesh=_mesh,
    scratch_types=[
        pltpu.VMEM((GA, CH), jnp.int32),
        pltpu.VMEM((GA, CH), jnp.int32),
        pltpu.VMEM((CH,), jnp.int32),
        pltpu.VMEM((CH,), jnp.int32),
        pltpu.VMEM((CH,), jnp.float32),
        pltpu.VMEM((SA,), jnp.float32),
        pltpu.VMEM((CH, 32), jnp.float32),
        pltpu.VMEM((CH, 32), jnp.float32),
        pltpu.VMEM_SHARED((AU, 32), jnp.float32),
        pltpu.VMEM_SHARED((AU,), jnp.float32),
        pltpu.SemaphoreType.DMA,
        pltpu.SemaphoreType.DMA,
        pltpu.SemaphoreType.DMA,
        pltpu.SemaphoreType.DMA,
        pltpu.SemaphoreType.DMA,
    ],
    compiler_params=_sc_params,
)(_sc_spmm_u_body)


def _sc_spmm_i_body(rows_g_ref, cols_s_ref, su_ref,
                    hi_ref,
                    ridx_blk, cidx_blk, gidx_a, gidx_b,
                    rowsv_a, rowsv_b,
                    acc, semga, semgb, semsa, semsb):
    c = lax.axis_index("c")
    s = lax.axis_index("s")
    gidx = (gidx_a, gidx_b)
    rowsv = (rowsv_a, rowsv_b)
    gsem = (semga, semgb)
    ssem = (semsa, semsb)
    for j in range(2):
        sl = c * 2 + j

        def _zrow(r, _):
            rowsv_a[r, pl.ds(0, 16)] = jnp.zeros((16,), jnp.float32)
            return 0
        lax.fori_loop(0, CH, _zrow, 0)

        for q in range(5):
            pltpu.sync_copy(rowsv_a, acc.at[pl.ds(s * 4500 + q * CH, CH)])
        pltpu.sync_copy(rowsv_a.at[pl.ds(0, 500)],
                        acc.at[pl.ds(s * 4500 + 5 * CH, 500)])
        plsc.subcore_barrier()
        soff = sl

        def _outer(t, _):
            blk = s * (EW // CH) + t * GB
            pltpu.sync_copy(rows_g_ref.at[pl.ds(blk, GB)], ridx_blk)
            pltpu.sync_copy(cols_s_ref.at[pl.ds(blk, GB)], cidx_blk)

            def _gidx(q, dst):
                def _off(w, __):
                    dst[pl.ds(w * 16, 16)] = (
                        ridx_blk[q, pl.ds(w * 16, 16)] * 4 + soff)
                    return 0
                lax.fori_loop(0, CH // 16, _off, 0)

            gd = [None, None]
            sd = [None, None]
            _gidx(0, gidx[0])
            gd[0] = pltpu.async_copy(su_ref.at[gidx[0]], rowsv[0], gsem[0])
            for q in range(GB):
                p = q & 1
                pn = 1 - p
                if q + 1 < GB:
                    if sd[pn] is not None:
                        sd[pn].wait()
                        sd[pn] = None
                    _gidx(q + 1, gidx[pn])
                    gd[pn] = pltpu.async_copy(
                        su_ref.at[gidx[pn]], rowsv[pn], gsem[pn])
                gd[p].wait()
                sd[p] = pltpu.async_copy(
                    rowsv[p], acc.at[cidx_blk.at[q]], ssem[p], add=True)
            for b in range(2):
                if sd[b] is not None:
                    sd[b].wait()
            return 0
        lax.fori_loop(0, TB, _outer, 0)
        plsc.subcore_barrier()

        def _wb(colo):
            for off, sz in ((0, CH), (CH, CH), (2 * CH, CH), (3 * CH, CH),
                            (4 * CH, CH), (5 * CH, 375)):
                pltpu.sync_copy(acc.at[pl.ds(s * 4375 + off, sz)],
                                rowsv_a.at[pl.ds(0, sz)])
                pltpu.sync_copy(
                    rowsv_a.at[pl.ds(0, sz)],
                    hi_ref.at[pl.ds(s * 4375 + off, sz), pl.ds(colo, 16)])

        @pl.when(c == 0)
        def _wb0():
            _wb(16 * j)

        @pl.when(c == 1)
        def _wb1():
            _wb(16 * (2 + j))


_sc_spmm_i = functools.partial(
    pl.kernel,
    out_type=jax.ShapeDtypeStruct((NI, D), jnp.float32),
    mesh=_mesh,
    scratch_types=[
        pltpu.VMEM((GB, CH), jnp.int32),
        pltpu.VMEM((GB, CH), jnp.int32),
        pltpu.VMEM((CH,), jnp.int32),
        pltpu.VMEM((CH,), jnp.int32),
        pltpu.VMEM((CH, 16), jnp.float32),
        pltpu.VMEM((CH, 16), jnp.float32),
        pltpu.VMEM_SHARED((AI, 16), jnp.float32),
        pltpu.SemaphoreType.DMA,
        pltpu.SemaphoreType.DMA,
        pltpu.SemaphoreType.DMA,
        pltpu.SemaphoreType.DMA,
    ],
    compiler_params=_sc_params,
)(_sc_spmm_i_body)


def _sc_gather_body(u_ref, i_ref, bu_ref, bp_ref, bn_ref,
                    ug_ref, pg_ref, ng_ref,
                    idxv, rowsv, sem):
    c = lax.axis_index("c")
    s = lax.axis_index("s")
    n = NB // (NC * NS)
    base = (s * NC + c) * n
    for src, idx_hbm, out in ((u_ref, bu_ref, ug_ref),
                              (i_ref, bp_ref, pg_ref),
                              (i_ref, bn_ref, ng_ref)):
        pltpu.sync_copy(idx_hbm.at[pl.ds(base, n)], idxv)
        pltpu.async_copy(src.at[idxv], rowsv, sem).wait()
        pltpu.sync_copy(rowsv, out.at[pl.ds(base, n)])


_sc_gather = functools.partial(
    pl.kernel,
    out_type=[jax.ShapeDtypeStruct((NB, D), jnp.float32)] * 3,
    mesh=_mesh,
    scratch_types=[
        pltpu.VMEM((NB // (NC * NS),), jnp.int32),
        pltpu.VMEM((NB // (NC * NS), D), jnp.float32),
        pltpu.SemaphoreType.DMA,
    ],
    compiler_params=_sc_params,
)(_sc_gather_body)


def _t1_body(re_ref, mask_ref, ent_ref, re2_ref, mask2_ref, ent2_ref, w2_ref,
             rfi_ref, rfu_ref):
    def _rel(r, m, e):
        x = r[...]
        x = jnp.exp(x - jnp.max(x, axis=1, keepdims=True))
        sm = x / jnp.sum(x, axis=1, keepdims=True)
        return jnp.dot(sm * m[...], e[...], preferred_element_type=jnp.float32)
    rfi_ref[...] = jnp.dot(_rel(re_ref, mask_ref, ent_ref), w2_ref[...],
                           preferred_element_type=jnp.float32)
    rfu_ref[...] = jnp.dot(_rel(re2_ref, mask2_ref, ent2_ref), w2_ref[...],
                           preferred_element_type=jnp.float32)


def _t1(re, mask, ent, re2, mask2, ent2, w2):
    return pl.pallas_call(
        _t1_body,
        out_shape=[jax.ShapeDtypeStruct((16, D), jnp.float32),
                   jax.ShapeDtypeStruct((8, D), jnp.float32)],
    )(re, mask, ent, re2, mask2, ent2, w2)


_BLK = 2000


def _t2_body(sn_ref, ir_ref, w1_ref, rfi_ref, out_ref):
    out_ref[...] = (
        jnp.dot(sn_ref[...], w1_ref[...], preferred_element_type=jnp.float32)
        + jnp.dot(ir_ref[...], rfi_ref[...], preferred_element_type=jnp.float32))


def _t2(struct_node_emb, ir, w1, rfi):
    nb = NI // _BLK
    return pl.pallas_call(
        _t2_body,
        grid=(nb,),
        in_specs=[
            pl.BlockSpec((_BLK, D), lambda i: (NU // _BLK + i, 0)),
            pl.BlockSpec((_BLK, 16), lambda i: (i, 0)),
            pl.BlockSpec((D, D), lambda i: (0, 0)),
            pl.BlockSpec((16, D), lambda i: (0, 0)),
        ],
        out_specs=pl.BlockSpec((_BLK, D), lambda i: (i, 0)),
        out_shape=jax.ShapeDtypeStruct((NI, D), jnp.float32),
    )(struct_node_emb, ir, w1, rfi)


def _t3_body(sn_ref, ur_ref, w1_ref, rfu_ref, out_ref):
    out_ref[...] = (
        jnp.dot(sn_ref[...], w1_ref[...], preferred_element_type=jnp.float32)
        + jnp.dot(ur_ref[...], rfu_ref[...], preferred_element_type=jnp.float32))


def _t3(struct_node_emb, ur, w1, rfu):
    nb = NU // _BLK
    return pl.pallas_call(
        _t3_body,
        grid=(nb,),
        in_specs=[
            pl.BlockSpec((_BLK, D), lambda i: (i, 0)),
            pl.BlockSpec((_BLK, 8), lambda i: (i, 0)),
            pl.BlockSpec((D, D), lambda i: (0, 0)),
            pl.BlockSpec((8, D), lambda i: (0, 0)),
        ],
        out_specs=pl.BlockSpec((_BLK, D), lambda i: (i, 0)),
        out_shape=jax.ShapeDtypeStruct((NU, D), jnp.float32),
    )(struct_node_emb, ur, w1, rfu)


def _t4_body(uc_ref, d0_ref, d1_ref, out_ref):
    deg = d0_ref[...] + d1_ref[...]
    dinv = jnp.where(deg > 0, 1.0 / deg, 0.0)
    out_ref[...] = uc_ref[...] * dinv


def _t4(user_comb, deg0, deg1):
    nb = NU // _BLK
    return pl.pallas_call(
        _t4_body,
        grid=(nb,),
        in_specs=[
            pl.BlockSpec((_BLK, D), lambda i: (i, 0)),
            pl.BlockSpec((_BLK, 1), lambda i: (i, 0)),
            pl.BlockSpec((_BLK, 1), lambda i: (i, 0)),
        ],
        out_specs=pl.BlockSpec((_BLK, D), lambda i: (i, 0)),
        out_shape=jax.ShapeDtypeStruct((NU, D), jnp.float32),
    )(user_comb, deg0, deg1)


def _finalize(x):
    x = jnp.where(x > 0, x, 0.2 * x)
    n = jnp.sqrt(jnp.sum(x * x, axis=1, keepdims=True))
    return NLAYERS * x / jnp.maximum(n, 1e-12)


def _t5u_body(a_ref, d0_ref, d1_ref, bias_ref, out_ref):
    deg = d0_ref[...] + d1_ref[...]
    dinv = jnp.where(deg > 0, 1.0 / deg, 0.0)
    out_ref[...] = _finalize(a_ref[...] * dinv + bias_ref[...])


def _t5u(hu, deg0, deg1, bias):
    nb = NU // _BLK
    return pl.pallas_call(
        _t5u_body,
        grid=(nb,),
        in_specs=[
            pl.BlockSpec((_BLK, D), lambda i: (i, 0)),
            pl.BlockSpec((_BLK, 1), lambda i: (i, 0)),
            pl.BlockSpec((_BLK, 1), lambda i: (i, 0)),
            pl.BlockSpec((_BLK, D), lambda i: (i, 0)),
        ],
        out_specs=pl.BlockSpec((_BLK, D), lambda i: (i, 0)),
        out_shape=jax.ShapeDtypeStruct((NU, D), jnp.float32),
    )(hu, deg0, deg1, bias)


def _t5i_body(h_ref, bias_ref, out_ref):
    out_ref[...] = _finalize(h_ref[...] + bias_ref[...])


def _t5i(hi, bias):
    nb = NI // _BLK
    return pl.pallas_call(
        _t5i_body,
        grid=(nb,),
        in_specs=[
            pl.BlockSpec((_BLK, D), lambda i: (i, 0)),
            pl.BlockSpec((_BLK, D), lambda i: (NU // _BLK + i, 0)),
        ],
        out_specs=pl.BlockSpec((_BLK, D), lambda i: (i, 0)),
        out_shape=jax.ShapeDtypeStruct((NI, D), jnp.float32),
    )(hi, bias)


def _t6_body(ug_ref, pg_ref, ng_ref, out_ref):
    ug, pg, ng = ug_ref[...], pg_ref[...], ng_ref[...]
    ps = jnp.sum(ug * pg, axis=1, keepdims=True)
    ns = jnp.sum(ug * ng, axis=1, keepdims=True)
    x = ps - ns
    ls = jnp.minimum(x, 0.0) - jnp.log(1.0 + jnp.exp(-jnp.abs(x)))
    mf = -jnp.sum(ls) / NB
    reg = (jnp.sum(ug * ug) + jnp.sum(pg * pg) + jnp.sum(ng * ng)) * 0.5
    out_ref[...] = jnp.reshape(mf + DECAY * reg / NB, (1, 1))


def _t6(ug, pg, ng):
    return pl.pallas_call(
        _t6_body,
        out_shape=jax.ShapeDtypeStruct((1, 1), jnp.float32),
    )(ug, pg, ng)


def kernel(struct_node_emb, train_weight, train_weight_2, bias, re, entity_emb,
           ir, re_2, entity_emb_2, ur, mask, mask_2, rows, cols,
           batch_users, batch_pos, batch_neg):
    rows = rows.astype(jnp.int32)
    cols = cols.astype(jnp.int32)
    pad = jnp.arange(NPAD, dtype=jnp.int32)
    blk2 = (EPAD // CH, CH)
    rows_s = jnp.concatenate([rows, NU + pad % NDUMU]).reshape(blk2)
    rows_g = jnp.concatenate([rows, pad % 16]).reshape(blk2)
    cols_s = jnp.concatenate([cols, NI + pad % NDUMI]).reshape(blk2)
    cols_g = jnp.concatenate([cols, pad % 16]).reshape(blk2)

    rfi, rfu = _t1(re, mask, entity_emb, re_2, mask_2, entity_emb_2,
                   train_weight_2)
    item_t = _t2(struct_node_emb, ir, train_weight, rfi).reshape(2 * NI, 32)
    user_comb = _t3(struct_node_emb, ur, train_weight, rfu)

    hu64, deg_flat = _sc_spmm_u(rows_s, cols_g, item_t)

    deg0 = deg_flat[:NU].reshape(NU, 1)
    deg1 = deg_flat[AU:AU + NU].reshape(NU, 1)
    su = _t4(user_comb, deg0, deg1).reshape(4 * NU, 16)

    hi64 = _sc_spmm_i(rows_g, cols_s, su)

    u = _t5u(hu64, deg0, deg1, bias)
    iv = _t5i(hi64, bias)

    ug, pg, ng = _sc_gather(u, iv, batch_users.astype(jnp.int32),
                            batch_pos.astype(jnp.int32),
                            batch_neg.astype(jnp.int32))
    loss = _t6(ug, pg, ng).reshape(())
    return (loss, u, iv)


# speedup vs baseline: 29.7029x; 1.0899x over previous
import functools

import jax
import jax.numpy as jnp
from jax import lax
from jax.experimental import pallas as pl
from jax.experimental.pallas import tpu as pltpu
from jax.experimental.pallas import tpu_sc as plsc

NU = 30000
NI = 70000
D = 64
NNZ = 1000000
NB = 4096
NLAYERS = 3.0
DECAY = 1e-4

NC = 2
NS = 16
CH = 800
EPAD = 1024000
NPAD = EPAD - NNZ
NDUMU = 704
NDUMI = 1600
AU = NU + 720
AI = NI + 2000
SA = AU // NS
EW = EPAD // NS
NCH = EW // CH
GA = 4
GB = 10
TA = NCH // GA
TB = NCH // GB

_mesh = plsc.VectorSubcoreMesh(core_axis_name="c", subcore_axis_name="s",
                               num_cores=NC, num_subcores=NS)
_sc_params = pltpu.CompilerParams(use_tc_tiling_on_sc=False)


def _sc_deg_body(rows_s_ref, deg_ref, didx_a, didx_b, onesv, zdeg, dacc, semd):
    c = lax.axis_index("c")
    s = lax.axis_index("s")
    didx = (didx_a, didx_b)

    def _init(t, _):
        onesv[pl.ds(t * 16, 16)] = jnp.full((16,), 1.0, jnp.float32)
        return 0
    lax.fori_loop(0, CH // 16, _init, 0)

    def _zdeg(t, _):
        zdeg[pl.ds(t * 16, 16)] = jnp.zeros((16,), jnp.float32)
        return 0
    lax.fori_loop(0, SA // 16, _zdeg, 0)

    pltpu.sync_copy(zdeg, dacc.at[pl.ds(s * SA, SA)])
    plsc.subcore_barrier()

    row0 = (c * NS + s) * (EPAD // (NC * NS) // CH)
    nrows = EPAD // (NC * NS) // CH
    sd = [None, None]
    for q in range(nrows):
        p = q & 1
        if sd[p] is not None:
            sd[p].wait()
        pltpu.sync_copy(rows_s_ref.at[pl.ds(row0 + q, 1)], didx[p])
        sd[p] = pltpu.async_copy(onesv, dacc.at[didx[p].at[0]], semd,
                                 add=True)
    for b in range(2):
        if sd[b] is not None:
            sd[b].wait()
    plsc.subcore_barrier()
    pltpu.sync_copy(dacc.at[pl.ds(s * SA, SA)], zdeg)
    pltpu.sync_copy(zdeg, deg_ref.at[pl.ds(c * AU + s * SA, SA)])


_sc_deg = functools.partial(
    pl.kernel,
    out_type=jax.ShapeDtypeStruct((NC * AU,), jnp.float32),
    mesh=_mesh,
    scratch_types=[
        pltpu.VMEM((1, CH), jnp.int32),
        pltpu.VMEM((1, CH), jnp.int32),
        pltpu.VMEM((CH,), jnp.float32),
        pltpu.VMEM((SA,), jnp.float32),
        pltpu.VMEM_SHARED((AU,), jnp.float32),
        pltpu.SemaphoreType.DMA,
    ],
    compiler_params=_sc_params,
)(_sc_deg_body)


def _sc_spmm_u_body(rows_s_ref, cols_g_ref, item_t_ref,
                    hu_ref,
                    cidx_blk, ridx_blk, gidx_a, gidx_b,
                    rowsv_a, rowsv_b,
                    acc, semga, semgb, semsa, semsb):
    c = lax.axis_index("c")
    s = lax.axis_index("s")
    gidx = (gidx_a, gidx_b)
    rowsv = (rowsv_a, rowsv_b)
    gsem = (semga, semgb)
    ssem = (semsa, semsb)

    def _zrow(r, _):
        rowsv_a[r, pl.ds(0, 16)] = jnp.zeros((16,), jnp.float32)
        rowsv_a[r, pl.ds(16, 16)] = jnp.zeros((16,), jnp.float32)
        return 0
    lax.fori_loop(0, CH, _zrow, 0)

    pltpu.sync_copy(rowsv_a, acc.at[pl.ds(s * SA, CH)])
    pltpu.sync_copy(rowsv_a, acc.at[pl.ds(s * SA + CH, CH)])
    pltpu.sync_copy(rowsv_a.at[pl.ds(0, SA - 2 * CH)],
                    acc.at[pl.ds(s * SA + 2 * CH, SA - 2 * CH)])
    plsc.subcore_barrier()

    coff = c

    def _outer(t, _):
        blk = s * (EW // CH) + t * GA
        pltpu.sync_copy(cols_g_ref.at[pl.ds(blk, GA)], cidx_blk)
        pltpu.sync_copy(rows_s_ref.at[pl.ds(blk, GA)], ridx_blk)

        def _gidx(q, dst):
            def _off(w, __):
                dst[pl.ds(w * 16, 16)] = (
                    cidx_blk[q, pl.ds(w * 16, 16)] * 2 + coff)
                return 0
            lax.fori_loop(0, CH // 16, _off, 0)

        gd = [None, None]
        sd = [None, None]
        _gidx(0, gidx[0])
        gd[0] = pltpu.async_copy(item_t_ref.at[gidx[0]], rowsv[0], gsem[0])
        for q in range(GA):
            p = q & 1
            pn = 1 - p
            if q + 1 < GA:
                if sd[pn] is not None:
                    sd[pn].wait()
                    sd[pn] = None
                _gidx(q + 1, gidx[pn])
                gd[pn] = pltpu.async_copy(
                    item_t_ref.at[gidx[pn]], rowsv[pn], gsem[pn])
            gd[p].wait()
            sd[p] = pltpu.async_copy(
                rowsv[p], acc.at[ridx_blk.at[q]], ssem[p], add=True)
        for b in range(2):
            if sd[b] is not None:
                sd[b].wait()
        return 0
    lax.fori_loop(0, TA, _outer, 0)
    plsc.subcore_barrier()

    def _wb(colo):
        for off, sz in ((0, CH), (CH, CH), (2 * CH, 275)):
            pltpu.sync_copy(acc.at[pl.ds(s * 1875 + off, sz)],
                            rowsv_a.at[pl.ds(0, sz)])
            pltpu.sync_copy(
                rowsv_a.at[pl.ds(0, sz)],
                hu_ref.at[pl.ds(s * 1875 + off, sz), pl.ds(colo, 32)])

    @pl.when(c == 0)
    def _wb0():
        _wb(0)

    @pl.when(c == 1)
    def _wb1():
        _wb(32)


_sc_spmm_u = functools.partial(
    pl.kernel,
    out_type=jax.ShapeDtypeStruct((NU, D), jnp.float32),
    mesh=_mesh,
    scratch_types=[
        pltpu.VMEM((GA, CH), jnp.int32),
        pltpu.VMEM((GA, CH), jnp.int32),
        pltpu.VMEM((CH,), jnp.int32),
        pltpu.VMEM((CH,), jnp.int32),
        pltpu.VMEM((CH, 32), jnp.float32),
        pltpu.VMEM((CH, 32), jnp.float32),
        pltpu.VMEM_SHARED((AU, 32), jnp.float32),
        pltpu.SemaphoreType.DMA,
        pltpu.SemaphoreType.DMA,
        pltpu.SemaphoreType.DMA,
        pltpu.SemaphoreType.DMA,
    ],
    compiler_params=_sc_params,
)(_sc_spmm_u_body)


def _sc_spmm_i_body(rows_g_ref, cols_s_ref, su_ref,
                    hi_ref,
                    ridx_blk, cidx_blk, gidx_a, gidx_b, gidx_c,
                    rowsv_a, rowsv_b, rowsv_c,
                    acc, semga, semgb, semgc, semsa, semsb, semsc):
    c = lax.axis_index("c")
    s = lax.axis_index("s")
    gidx = (gidx_a, gidx_b, gidx_c)
    rowsv = (rowsv_a, rowsv_b, rowsv_c)
    gsem = (semga, semgb, semgc)
    ssem = (semsa, semsb, semsc)
    for j in range(2):
        sl = c * 2 + j

        def _zrow(r, _):
            rowsv_a[r, pl.ds(0, 16)] = jnp.zeros((16,), jnp.float32)
            return 0
        lax.fori_loop(0, CH, _zrow, 0)

        for q in range(5):
            pltpu.sync_copy(rowsv_a, acc.at[pl.ds(s * 4500 + q * CH, CH)])
        pltpu.sync_copy(rowsv_a.at[pl.ds(0, 500)],
                        acc.at[pl.ds(s * 4500 + 5 * CH, 500)])
        plsc.subcore_barrier()
        soff = sl

        def _outer(t, _):
            blk = s * (EW // CH) + t * GB
            pltpu.sync_copy(rows_g_ref.at[pl.ds(blk, GB)], ridx_blk)
            pltpu.sync_copy(cols_s_ref.at[pl.ds(blk, GB)], cidx_blk)

            def _gidx(q, dst):
                def _off(w, __):
                    dst[pl.ds(w * 16, 16)] = (
                        ridx_blk[q, pl.ds(w * 16, 16)] * 4 + soff)
                    return 0
                lax.fori_loop(0, CH // 16, _off, 0)

            gd = [None, None, None]
            sd = [None, None, None]
            for q0 in range(2):
                _gidx(q0, gidx[q0])
                gd[q0] = pltpu.async_copy(
                    su_ref.at[gidx[q0]], rowsv[q0], gsem[q0])
            for q in range(GB):
                p = q % 3
                p2 = (q + 2) % 3
                if q + 2 < GB:
                    if sd[p2] is not None:
                        sd[p2].wait()
                        sd[p2] = None
                    _gidx(q + 2, gidx[p2])
                    gd[p2] = pltpu.async_copy(
                        su_ref.at[gidx[p2]], rowsv[p2], gsem[p2])
                gd[p].wait()
                sd[p] = pltpu.async_copy(
                    rowsv[p], acc.at[cidx_blk.at[q]], ssem[p], add=True)
            for b in range(3):
                if sd[b] is not None:
                    sd[b].wait()
            return 0
        lax.fori_loop(0, TB, _outer, 0)
        plsc.subcore_barrier()

        def _wb(colo):
            for off, sz in ((0, CH), (CH, CH), (2 * CH, CH), (3 * CH, CH),
                            (4 * CH, CH), (5 * CH, 375)):
                pltpu.sync_copy(acc.at[pl.ds(s * 4375 + off, sz)],
                                rowsv_a.at[pl.ds(0, sz)])
                pltpu.sync_copy(
                    rowsv_a.at[pl.ds(0, sz)],
                    hi_ref.at[pl.ds(s * 4375 + off, sz), pl.ds(colo, 16)])

        @pl.when(c == 0)
        def _wb0():
            _wb(16 * j)

        @pl.when(c == 1)
        def _wb1():
            _wb(16 * (2 + j))


_sc_spmm_i = functools.partial(
    pl.kernel,
    out_type=jax.ShapeDtypeStruct((NI, D), jnp.float32),
    mesh=_mesh,
    scratch_types=[
        pltpu.VMEM((GB, CH), jnp.int32),
        pltpu.VMEM((GB, CH), jnp.int32),
        pltpu.VMEM((CH,), jnp.int32),
        pltpu.VMEM((CH,), jnp.int32),
        pltpu.VMEM((CH,), jnp.int32),
        pltpu.VMEM((CH, 16), jnp.float32),
        pltpu.VMEM((CH, 16), jnp.float32),
        pltpu.VMEM((CH, 16), jnp.float32),
        pltpu.VMEM_SHARED((AI, 16), jnp.float32),
        pltpu.SemaphoreType.DMA,
        pltpu.SemaphoreType.DMA,
        pltpu.SemaphoreType.DMA,
        pltpu.SemaphoreType.DMA,
        pltpu.SemaphoreType.DMA,
        pltpu.SemaphoreType.DMA,
    ],
    compiler_params=_sc_params,
)(_sc_spmm_i_body)


def _sc_gather_body(u_ref, i_ref, bu_ref, bp_ref, bn_ref,
                    ug_ref, pg_ref, ng_ref,
                    idxv, rowsv, sem):
    c = lax.axis_index("c")
    s = lax.axis_index("s")
    n = NB // (NC * NS)
    base = (s * NC + c) * n
    for src, idx_hbm, out in ((u_ref, bu_ref, ug_ref),
                              (i_ref, bp_ref, pg_ref),
                              (i_ref, bn_ref, ng_ref)):
        pltpu.sync_copy(idx_hbm.at[pl.ds(base, n)], idxv)
        pltpu.async_copy(src.at[idxv], rowsv, sem).wait()
        pltpu.sync_copy(rowsv, out.at[pl.ds(base, n)])


_sc_gather = functools.partial(
    pl.kernel,
    out_type=[jax.ShapeDtypeStruct((NB, D), jnp.float32)] * 3,
    mesh=_mesh,
    scratch_types=[
        pltpu.VMEM((NB // (NC * NS),), jnp.int32),
        pltpu.VMEM((NB // (NC * NS), D), jnp.float32),
        pltpu.SemaphoreType.DMA,
    ],
    compiler_params=_sc_params,
)(_sc_gather_body)


def _t1_body(re_ref, mask_ref, ent_ref, re2_ref, mask2_ref, ent2_ref, w2_ref,
             rfi_ref, rfu_ref):
    def _rel(r, m, e):
        x = r[...]
        x = jnp.exp(x - jnp.max(x, axis=1, keepdims=True))
        sm = x / jnp.sum(x, axis=1, keepdims=True)
        return jnp.dot(sm * m[...], e[...], preferred_element_type=jnp.float32)
    rfi_ref[...] = jnp.dot(_rel(re_ref, mask_ref, ent_ref), w2_ref[...],
                           preferred_element_type=jnp.float32)
    rfu_ref[...] = jnp.dot(_rel(re2_ref, mask2_ref, ent2_ref), w2_ref[...],
                           preferred_element_type=jnp.float32)


def _t1(re, mask, ent, re2, mask2, ent2, w2):
    return pl.pallas_call(
        _t1_body,
        out_shape=[jax.ShapeDtypeStruct((16, D), jnp.float32),
                   jax.ShapeDtypeStruct((8, D), jnp.float32)],
    )(re, mask, ent, re2, mask2, ent2, w2)


_BLK = 2000


def _t2_body(sn_ref, ir_ref, w1_ref, rfi_ref, out_ref):
    out_ref[...] = (
        jnp.dot(sn_ref[...], w1_ref[...], preferred_element_type=jnp.float32)
        + jnp.dot(ir_ref[...], rfi_ref[...], preferred_element_type=jnp.float32))


def _t2(struct_node_emb, ir, w1, rfi):
    nb = NI // _BLK
    return pl.pallas_call(
        _t2_body,
        grid=(nb,),
        in_specs=[
            pl.BlockSpec((_BLK, D), lambda i: (NU // _BLK + i, 0)),
            pl.BlockSpec((_BLK, 16), lambda i: (i, 0)),
            pl.BlockSpec((D, D), lambda i: (0, 0)),
            pl.BlockSpec((16, D), lambda i: (0, 0)),
        ],
        out_specs=pl.BlockSpec((_BLK, D), lambda i: (i, 0)),
        out_shape=jax.ShapeDtypeStruct((NI, D), jnp.float32),
    )(struct_node_emb, ir, w1, rfi)


def _t3_body(sn_ref, ur_ref, w1_ref, rfu_ref, out_ref):
    out_ref[...] = (
        jnp.dot(sn_ref[...], w1_ref[...], preferred_element_type=jnp.float32)
        + jnp.dot(ur_ref[...], rfu_ref[...], preferred_element_type=jnp.float32))


def _t3(struct_node_emb, ur, w1, rfu):
    nb = NU // _BLK
    return pl.pallas_call(
        _t3_body,
        grid=(nb,),
        in_specs=[
            pl.BlockSpec((_BLK, D), lambda i: (i, 0)),
            pl.BlockSpec((_BLK, 8), lambda i: (i, 0)),
            pl.BlockSpec((D, D), lambda i: (0, 0)),
            pl.BlockSpec((8, D), lambda i: (0, 0)),
        ],
        out_specs=pl.BlockSpec((_BLK, D), lambda i: (i, 0)),
        out_shape=jax.ShapeDtypeStruct((NU, D), jnp.float32),
    )(struct_node_emb, ur, w1, rfu)


def _t4_body(uc_ref, d0_ref, d1_ref, out_ref):
    deg = d0_ref[...] + d1_ref[...]
    dinv = jnp.where(deg > 0, 1.0 / deg, 0.0)
    out_ref[...] = uc_ref[...] * dinv


def _t4(user_comb, deg0, deg1):
    nb = NU // _BLK
    return pl.pallas_call(
        _t4_body,
        grid=(nb,),
        in_specs=[
            pl.BlockSpec((_BLK, D), lambda i: (i, 0)),
            pl.BlockSpec((_BLK, 1), lambda i: (i, 0)),
            pl.BlockSpec((_BLK, 1), lambda i: (i, 0)),
        ],
        out_specs=pl.BlockSpec((_BLK, D), lambda i: (i, 0)),
        out_shape=jax.ShapeDtypeStruct((NU, D), jnp.float32),
    )(user_comb, deg0, deg1)


def _finalize(x):
    x = jnp.where(x > 0, x, 0.2 * x)
    n = jnp.sqrt(jnp.sum(x * x, axis=1, keepdims=True))
    return NLAYERS * x / jnp.maximum(n, 1e-12)


def _t5u_body(a_ref, d0_ref, d1_ref, bias_ref, out_ref):
    deg = d0_ref[...] + d1_ref[...]
    dinv = jnp.where(deg > 0, 1.0 / deg, 0.0)
    out_ref[...] = _finalize(a_ref[...] * dinv + bias_ref[...])


def _t5u(hu, deg0, deg1, bias):
    nb = NU // _BLK
    return pl.pallas_call(
        _t5u_body,
        grid=(nb,),
        in_specs=[
            pl.BlockSpec((_BLK, D), lambda i: (i, 0)),
            pl.BlockSpec((_BLK, 1), lambda i: (i, 0)),
            pl.BlockSpec((_BLK, 1), lambda i: (i, 0)),
            pl.BlockSpec((_BLK, D), lambda i: (i, 0)),
        ],
        out_specs=pl.BlockSpec((_BLK, D), lambda i: (i, 0)),
        out_shape=jax.ShapeDtypeStruct((NU, D), jnp.float32),
    )(hu, deg0, deg1, bias)


def _t5i_body(h_ref, bias_ref, out_ref):
    out_ref[...] = _finalize(h_ref[...] + bias_ref[...])


def _t5i(hi, bias):
    nb = NI // _BLK
    return pl.pallas_call(
        _t5i_body,
        grid=(nb,),
        in_specs=[
            pl.BlockSpec((_BLK, D), lambda i: (i, 0)),
            pl.BlockSpec((_BLK, D), lambda i: (NU // _BLK + i, 0)),
        ],
        out_specs=pl.BlockSpec((_BLK, D), lambda i: (i, 0)),
        out_shape=jax.ShapeDtypeStruct((NI, D), jnp.float32),
    )(hi, bias)


def _t6_body(ug_ref, pg_ref, ng_ref, out_ref):
    ug, pg, ng = ug_ref[...], pg_ref[...], ng_ref[...]
    ps = jnp.sum(ug * pg, axis=1, keepdims=True)
    ns = jnp.sum(ug * ng, axis=1, keepdims=True)
    x = ps - ns
    ls = jnp.minimum(x, 0.0) - jnp.log(1.0 + jnp.exp(-jnp.abs(x)))
    mf = -jnp.sum(ls) / NB
    reg = (jnp.sum(ug * ug) + jnp.sum(pg * pg) + jnp.sum(ng * ng)) * 0.5
    out_ref[...] = jnp.reshape(mf + DECAY * reg / NB, (1, 1))


def _t6(ug, pg, ng):
    return pl.pallas_call(
        _t6_body,
        out_shape=jax.ShapeDtypeStruct((1, 1), jnp.float32),
    )(ug, pg, ng)


def kernel(struct_node_emb, train_weight, train_weight_2, bias, re, entity_emb,
           ir, re_2, entity_emb_2, ur, mask, mask_2, rows, cols,
           batch_users, batch_pos, batch_neg):
    rows = rows.astype(jnp.int32)
    cols = cols.astype(jnp.int32)
    pad = jnp.arange(NPAD, dtype=jnp.int32)
    blk2 = (EPAD // CH, CH)
    rows_s = jnp.concatenate([rows, NU + pad % NDUMU]).reshape(blk2)
    rows_g = jnp.concatenate([rows, pad % 16]).reshape(blk2)
    cols_s = jnp.concatenate([cols, NI + pad % NDUMI]).reshape(blk2)
    cols_g = jnp.concatenate([cols, pad % 16]).reshape(blk2)

    rfi, rfu = _t1(re, mask, entity_emb, re_2, mask_2, entity_emb_2,
                   train_weight_2)
    item_t = _t2(struct_node_emb, ir, train_weight, rfi).reshape(2 * NI, 32)
    user_comb = _t3(struct_node_emb, ur, train_weight, rfu)

    deg_flat = _sc_deg(rows_s)
    hu64 = _sc_spmm_u(rows_s, cols_g, item_t)

    deg0 = deg_flat[:NU].reshape(NU, 1)
    deg1 = deg_flat[AU:AU + NU].reshape(NU, 1)
    su = _t4(user_comb, deg0, deg1).reshape(4 * NU, 16)

    hi64 = _sc_spmm_i(rows_g, cols_s, su)

    u = _t5u(hu64, deg0, deg1, bias)
    iv = _t5i(hi64, bias)

    ug, pg, ng = _sc_gather(u, iv, batch_users.astype(jnp.int32),
                            batch_pos.astype(jnp.int32),
                            batch_neg.astype(jnp.int32))
    loss = _t6(ug, pg, ng).reshape(())
    return (loss, u, iv)
